# baseline (speedup 1.0000x reference)
import functools
import math

import jax
import jax.numpy as jnp
from jax import lax
from jax.experimental import pallas as pl
from jax.experimental.pallas import tpu as pltpu
from jax.experimental.pallas import tpu_sc as plsc

EPS = 1e-5
N = 10000
E = 320000
D_IN = 128
D_HID = 128
D_EMB = 192
D_FACT = 32
D_EMBEDS = 64
M = 100000
TEMP = 0.2
CL_B = 4096

NC = 2
NS = 16
NW = NC * NS

EPW = E // NW
ECH = 80
NECH = EPW // ECH
MCH = 80
NMCH = M // MCH

def _mesh():
    return plsc.VectorSubcoreMesh(core_axis_name="c", subcore_axis_name="s")


_RCH = 80
_NRCH = N // _RCH


def _chunk_loop(nchunks, fn):
    s = lax.axis_index("s")

    def body(u, carry):
        j = s + u * NS

        @pl.when(j < nchunks)
        def _():
            fn(j)
        return carry

    lax.fori_loop(0, (nchunks + NS - 1) // NS, body, 0)


@functools.lru_cache(maxsize=None)
def _build_deg():
    @functools.partial(
        pl.kernel, mesh=_mesh(),
        out_type=jax.ShapeDtypeStruct((NC, N, 128), jnp.float32),
        scratch_types=[
            pltpu.VMEM((ECH,), jnp.int32),
            pltpu.VMEM((ECH,), jnp.int32),
            pltpu.VMEM((ECH, 128), jnp.float32),
            pltpu.VMEM((_RCH, 128), jnp.float32),
            pltpu.VMEM_SHARED((N, 128), jnp.float32),
            pltpu.SemaphoreType.DMA,
            pltpu.SemaphoreType.DMA,
            pltpu.SemaphoreType.DMA,
            pltpu.SemaphoreType.DMA,
        ],
    )
    def k(dst_hbm, ones_h, zeros_h, out_hbm, didx0, didx1, ones_v, zbuf,
          table, ds0, ds1, ss0, ss1):
        c = lax.axis_index("c")
        s = lax.axis_index("s")
        wid = s * NC + c
        pltpu.sync_copy(zeros_h.at[pl.ds(0, _RCH)], zbuf)
        _chunk_loop(_NRCH,
                    lambda j: pltpu.sync_copy(zbuf, table.at[pl.ds(j * _RCH, _RCH)]))
        pltpu.sync_copy(ones_h, ones_v)
        plsc.subcore_barrier()
        base = wid * EPW
        didxs = (didx0, didx1)
        dsem = (ds0, ds1)
        ssem = (ss0, ss1)
        pltpu.async_copy(dst_hbm.at[pl.ds(base, ECH)], didx0, ds0)

        def dbody(j, carry):
            for b in range(2):
                nb = 1 - b

                @pl.when(lax.rem(j, 2) == b)
                def _():
                    pltpu.make_async_copy(
                        dst_hbm.at[pl.ds(base, ECH)], didxs[b], dsem[b]).wait()
                    pltpu.async_copy(ones_v, table.at[didxs[b]], ssem[b],
                                     add=True)

                    @pl.when(j + 1 < NECH)
                    def _():
                        @pl.when(j > 0)
                        def _():
                            pltpu.make_async_copy(
                                ones_v, table.at[didxs[nb]], ssem[nb]).wait()
                        pltpu.async_copy(
                            dst_hbm.at[pl.ds(base + (j + 1) * ECH, ECH)],
                            didxs[nb], dsem[nb])
            return carry

        lax.fori_loop(0, NECH, dbody, 0)
        pltpu.make_async_copy(ones_v, table.at[didx0], ss0).wait()
        pltpu.make_async_copy(ones_v, table.at[didx1], ss1).wait()
        plsc.subcore_barrier()

        def out_chunk(j):
            rows = pl.ds(j * _RCH, _RCH)
            pltpu.sync_copy(table.at[rows], zbuf)
            pltpu.sync_copy(zbuf, out_hbm.at[c, rows])

        _chunk_loop(_NRCH, out_chunk)

    return k


def _sc_deg(dst_flat, ones_hbm, zeros_hbm):
    return _build_deg()(dst_flat, ones_hbm, zeros_hbm)


@functools.lru_cache(maxsize=None)
def _build_gather():
    nu = (NMCH + NW - 1) // NW

    @functools.partial(
        pl.kernel, mesh=_mesh(),
        out_type=jax.ShapeDtypeStruct((5, M, 128), jnp.float32),
        scratch_types=[
            pltpu.VMEM((MCH,), jnp.int32),
            pltpu.VMEM((MCH,), jnp.int32),
            pltpu.VMEM((MCH, 128), jnp.float32),
            pltpu.VMEM((MCH, 128), jnp.float32),
            pltpu.SemaphoreType.DMA,
            pltpu.SemaphoreType.DMA,
            pltpu.SemaphoreType.DMA,
            pltpu.SemaphoreType.DMA,
        ],
    )
    def k(ptab, i0, i1, i2, i3, i4, gout_hbm,
          idx0, idx1, rows0, rows1, gs0, gs1, ws0, ws1):
        c = lax.axis_index("c")
        s = lax.axis_index("s")
        wid = s * NC + c
        idxs = (idx0, idx1)
        rows = (rows0, rows1)
        gsem = (gs0, gs1)
        wsem = (ws0, ws1)
        for q, idx_hbm in enumerate((i0, i1, i2, i3, i4)):
            pltpu.sync_copy(idx_hbm.at[pl.ds(wid * MCH, MCH)], idx0)
            pltpu.async_copy(ptab.at[idx0], rows0, gs0)

            def body(u, carry):
                j = wid + u * NW
                jn = j + NW

                @pl.when(j < NMCH)
                def _():
                    for b in range(2):
                        nb = 1 - b

                        @pl.when(lax.rem(u, 2) == b)
                        def _():
                            pltpu.make_async_copy(
                                ptab.at[idxs[b]], rows[b], gsem[b]).wait()
                            pltpu.async_copy(
                                rows[b], gout_hbm.at[q, pl.ds(j * MCH, MCH)],
                                wsem[b])

                            @pl.when(jn < NMCH)
                            def _():
                                pltpu.sync_copy(
                                    idx_hbm.at[pl.ds(jn * MCH, MCH)], idxs[nb])

                                @pl.when(u > 0)
                                def _():
                                    pltpu.make_async_copy(
                                        rows[nb],
                                        gout_hbm.at[q, pl.ds(jn * MCH, MCH)],
                                        wsem[nb]).wait()
                                pltpu.async_copy(
                                    ptab.at[idxs[nb]], rows[nb], gsem[nb])
                return carry

            lax.fori_loop(0, nu, body, 0)
            pltpu.make_async_copy(rows0, gout_hbm.at[q, pl.ds(wid * MCH, MCH)],
                                  ws0).wait()
            pltpu.make_async_copy(rows1, gout_hbm.at[q, pl.ds(wid * MCH, MCH)],
                                  ws1).wait()

    return k


def _sc_gather(ptab, idxs):
    return _build_gather()(ptab, *idxs)


@functools.lru_cache(maxsize=None)
def _make_segsum(out_n, cpy):
    @functools.partial(
        pl.kernel, mesh=_mesh(),
        out_type=jax.ShapeDtypeStruct((NC, out_n, 128), jnp.float32),
        scratch_types=[
            pltpu.VMEM((EPW,), jnp.int32),
            pltpu.VMEM((ECH,), jnp.int32),
            pltpu.VMEM((ECH,), jnp.int32),
            pltpu.VMEM((ECH, 128), jnp.float32),
            pltpu.VMEM((ECH, 128), jnp.float32),
            pltpu.VMEM((128, 128), jnp.float32),
            pltpu.VMEM_SHARED((N, 128), jnp.float32),
            pltpu.SemaphoreType.DMA,
            pltpu.SemaphoreType.DMA,
            pltpu.SemaphoreType.DMA,
            pltpu.SemaphoreType.DMA,
            pltpu.SemaphoreType.DMA,
            pltpu.SemaphoreType.DMA,
        ],
    )
    def k(src_hbm, dst_hbm, vals_hbm, zeros_h, out_hbm,
          sidx, didx0, didx1, rows0, rows1, zbuf, table,
          sem0, sem1, ds0, ds1, ss0, ss1):
        c = lax.axis_index("c")
        s = lax.axis_index("s")
        wid = s * NC + c
        base = wid * EPW
        pltpu.sync_copy(src_hbm.at[pl.ds(base, EPW)], sidx)
        pltpu.sync_copy(zeros_h, zbuf)
        _chunk_loop(_NRCH,
                    lambda j: pltpu.sync_copy(zbuf.at[pl.ds(0, _RCH)],
                                              table.at[pl.ds(j * _RCH, _RCH)]))
        plsc.subcore_barrier()

        didxs = (didx0, didx1)
        rows = (rows0, rows1)
        gsem = (sem0, sem1)
        dsem = (ds0, ds1)
        ssem = (ss0, ss1)
        pltpu.async_copy(vals_hbm.at[sidx.at[pl.ds(0, ECH)]], rows0, sem0)
        pltpu.async_copy(dst_hbm.at[pl.ds(base, ECH)], didx0, ds0)

        def body(j, carry):
            for b in range(2):
                nb = 1 - b

                @pl.when(lax.rem(j, 2) == b)
                def _():
                    pltpu.make_async_copy(
                        dst_hbm.at[pl.ds(base, ECH)], didxs[b], dsem[b]).wait()
                    pltpu.make_async_copy(
                        vals_hbm.at[sidx.at[pl.ds(0, ECH)]], rows[b],
                        gsem[b]).wait()
                    pltpu.async_copy(rows[b], table.at[didxs[b]], ssem[b],
                                     add=True)

                    @pl.when(j + 1 < NECH)
                    def _():
                        @pl.when(j > 0)
                        def _():
                            pltpu.make_async_copy(
                                rows[nb], table.at[didxs[nb]], ssem[nb]).wait()
                        pltpu.async_copy(
                            vals_hbm.at[sidx.at[pl.ds((j + 1) * ECH, ECH)]],
                            rows[nb], gsem[nb])
                        pltpu.async_copy(
                            dst_hbm.at[pl.ds(base + (j + 1) * ECH, ECH)],
                            didxs[nb], dsem[nb])
            return carry

        lax.fori_loop(0, NECH, body, 0)
        pltpu.make_async_copy(rows0, table.at[didx0], ss0).wait()
        pltpu.make_async_copy(rows1, table.at[didx1], ss1).wait()
        plsc.subcore_barrier()

        def out_chunk(j):
            rows = pl.ds(j * cpy, cpy)
            pltpu.sync_copy(table.at[rows], zbuf.at[pl.ds(0, cpy)])
            pltpu.sync_copy(zbuf.at[pl.ds(0, cpy)], out_hbm.at[c, rows])

        _chunk_loop(out_n // cpy, out_chunk)

    return k


def _segsum_full(src_flat, dst_flat, vals, zeros_hbm):
    return _make_segsum(N, 80)(src_flat, dst_flat, vals, zeros_hbm)


def _segsum_cl(src_flat, dst_flat, vals, zeros_hbm):
    return _make_segsum(CL_B, 64)(src_flat, dst_flat, vals, zeros_hbm)


_R1 = 1000


def _prep_body(r0, r1, r2, p0_ref, p1_ref, pt_ref):
    ps = []
    for (r_ref, p_ref, kk) in ((r0, p0_ref, 0.5), (r1, p1_ref, -0.3)):
        f = r_ref[...]
        radius = 1.0 / math.sqrt(abs(kk))
        nrm = jnp.sqrt(jnp.sum(f * f, axis=-1, keepdims=True)) + EPS
        p = f / nrm * (0.45 * radius)
        p_ref[...] = p
        ps.append(p)
    ps.append(r2[...])
    ps.append(jnp.zeros((ps[0].shape[0], 32), jnp.float32))
    pt_ref[...] = jnp.concatenate(ps, axis=-1)


_tc_prep = pl.pallas_call(
    _prep_body,
    grid=(N // _R1,),
    in_specs=[
        pl.BlockSpec((_R1, 32), lambda i: (i, 0)),
        pl.BlockSpec((_R1, 32), lambda i: (i, 0)),
        pl.BlockSpec((_R1, 32), lambda i: (i, 0)),
    ],
    out_specs=[
        pl.BlockSpec((_R1, 32), lambda i: (i, 0)),
        pl.BlockSpec((_R1, 32), lambda i: (i, 0)),
        pl.BlockSpec((_R1, 128), lambda i: (i, 0)),
    ],
    out_shape=[
        jax.ShapeDtypeStruct((N, 32), jnp.float32),
        jax.ShapeDtypeStruct((N, 32), jnp.float32),
        jax.ShapeDtypeStruct((N, 128), jnp.float32),
    ],
)


def _dis_of(dega, degb):
    deg = dega[:, 0:1] + degb[:, 0:1]
    return 1.0 / jnp.sqrt(jnp.maximum(deg, 1.0))


def _xs_body(dega, degb, x_ref, xs_ref):
    xs_ref[...] = x_ref[...] * _dis_of(dega[...], degb[...])


_tc_xs = pl.pallas_call(
    _xs_body,
    grid=(N // _R1,),
    in_specs=[
        pl.BlockSpec((_R1, 128), lambda i: (i, 0)),
        pl.BlockSpec((_R1, 128), lambda i: (i, 0)),
        pl.BlockSpec((_R1, 128), lambda i: (i, 0)),
    ],
    out_specs=pl.BlockSpec((_R1, 128), lambda i: (i, 0)),
    out_shape=jax.ShapeDtypeStruct((N, 128), jnp.float32),
)


def _layer1_body(g1a, g1b, dega, degb, w1, b1, out_ref):
    dis = _dis_of(dega[...], degb[...])
    g = (g1a[...] + g1b[...]) * dis
    h = jnp.dot(g, w1[...], preferred_element_type=jnp.float32) + b1[...]
    out_ref[...] = jnp.maximum(h, 0.0) * dis


_tc_layer1 = pl.pallas_call(
    _layer1_body,
    grid=(N // _R1,),
    in_specs=[
        pl.BlockSpec((_R1, 128), lambda i: (i, 0)),
        pl.BlockSpec((_R1, 128), lambda i: (i, 0)),
        pl.BlockSpec((_R1, 128), lambda i: (i, 0)),
        pl.BlockSpec((_R1, 128), lambda i: (i, 0)),
        pl.BlockSpec((128, 128), lambda i: (0, 0)),
        pl.BlockSpec((1, 128), lambda i: (0, 0)),
    ],
    out_specs=pl.BlockSpec((_R1, 128), lambda i: (i, 0)),
    out_shape=jax.ShapeDtypeStruct((N, 128), jnp.float32),
)


_R2 = 512
_NB = CL_B // _R2


def _lap_feats(p, w, b, kk):
    pw = lax.dot_general(p, w, (((1,), (1,)), ((), ())),
                         preferred_element_type=jnp.float32)
    if kk == 0.0:
        dist = pw
    else:
        xx = jnp.sum(p * p, axis=-1, keepdims=True)
        ww = jnp.sum(w * w, axis=-1)[None, :]
        div = xx - 2.0 * pw + ww
        dist = jnp.log((1.0 + kk * xx) / (div + EPS))
    return jnp.exp((D_FACT - 1) * dist / 2.0) * jnp.cos(dist + b)


def _cl_body(g2a, g2b, dega, degb, w2, bias2v, p0, p1, p2,
             ws0, ws1, ws2, bs0, bs1, bs2,
             rs_out, cs_out, ps_out, h4s, laps, rs, cs, ps):
    i = pl.program_id(0)
    j = pl.program_id(1)

    @pl.when(j == 0)
    def _():
        dis = _dis_of(dega[...], degb[...])
        g = (g2a[...] + g2b[...]) * dis
        h4s[...] = (jnp.dot(g, w2[...], preferred_element_type=jnp.float32)
                    + bias2v[...])

    @pl.when(i == 0)
    def _():
        laps[pl.ds(j * _R2, _R2), :] = jnp.concatenate(
            [_lap_feats(p0[...], ws0[...], bs0[...], 0.5),
             _lap_feats(p1[...], ws1[...], bs1[...], -0.3),
             _lap_feats(p2[...], ws2[...], bs2[...], 0.0)], axis=-1)

    hb = h4s[...]
    lb = laps[pl.ds(j * _R2, _R2), :]
    n1 = jnp.sqrt(jnp.sum(hb * hb, axis=-1, keepdims=True))
    n2 = jnp.sqrt(jnp.sum(lb * lb, axis=-1))[None, :]
    d = lax.dot_general(hb, lb, (((1,), (1,)), ((), ())),
                        preferred_element_type=jnp.float32)
    s = jnp.exp(d / (n1 * n2 + EPS) / TEMP)
    rowv = jnp.sum(s, axis=1)[None, :]
    colv = jnp.sum(s, axis=0)[None, :]

    @pl.when(j == 0)
    def _():
        rs[pl.ds(i, 1), :] = rowv

    @pl.when(j != 0)
    def _():
        rs[pl.ds(i, 1), :] += rowv

    @pl.when(i == 0)
    def _():
        cs[pl.ds(j, 1), :] = colv

    @pl.when(i != 0)
    def _():
        cs[pl.ds(j, 1), :] += colv

    @pl.when(i == j)
    def _():
        rr = lax.broadcasted_iota(jnp.int32, (_R2, _R2), 0)
        cc = lax.broadcasted_iota(jnp.int32, (_R2, _R2), 1)
        diag = jnp.sum(jnp.where(rr == cc, s, 0.0), axis=1)[None, :]
        ps[pl.ds(i, 1), :] = diag

    @pl.when((i == _NB - 1) & (j == _NB - 1))
    def _():
        rs_out[...] = rs[...]
        cs_out[...] = cs[...]
        ps_out[...] = ps[...]


_tc_cl = pl.pallas_call(
    _cl_body,
    grid=(_NB, _NB),
    in_specs=[
        pl.BlockSpec((_R2, 128), lambda i, j: (i, 0)),
        pl.BlockSpec((_R2, 128), lambda i, j: (i, 0)),
        pl.BlockSpec((_R2, 128), lambda i, j: (i, 0)),
        pl.BlockSpec((_R2, 128), lambda i, j: (i, 0)),
        pl.BlockSpec((128, 192), lambda i, j: (0, 0)),
        pl.BlockSpec((1, 192), lambda i, j: (0, 0)),
        pl.BlockSpec((_R2, 32), lambda i, j: (j, 0)),
        pl.BlockSpec((_R2, 32), lambda i, j: (j, 0)),
        pl.BlockSpec((_R2, 32), lambda i, j: (j, 0)),
        pl.BlockSpec((64, 32), lambda i, j: (0, 0)),
        pl.BlockSpec((64, 32), lambda i, j: (0, 0)),
        pl.BlockSpec((64, 32), lambda i, j: (0, 0)),
        pl.BlockSpec((1, 64), lambda i, j: (0, 0)),
        pl.BlockSpec((1, 64), lambda i, j: (0, 0)),
        pl.BlockSpec((1, 64), lambda i, j: (0, 0)),
    ],
    out_specs=[
        pl.BlockSpec((_NB, _R2), lambda i, j: (0, 0)),
        pl.BlockSpec((_NB, _R2), lambda i, j: (0, 0)),
        pl.BlockSpec((_NB, _R2), lambda i, j: (0, 0)),
    ],
    out_shape=[
        jax.ShapeDtypeStruct((_NB, _R2), jnp.float32),
        jax.ShapeDtypeStruct((_NB, _R2), jnp.float32),
        jax.ShapeDtypeStruct((_NB, _R2), jnp.float32),
    ],
    scratch_shapes=[
        pltpu.VMEM((_R2, 192), jnp.float32),
        pltpu.VMEM((CL_B, 192), jnp.float32),
        pltpu.VMEM((_NB, _R2), jnp.float32),
        pltpu.VMEM((_NB, _R2), jnp.float32),
        pltpu.VMEM((_NB, _R2), jnp.float32),
    ],
)


_B6 = 2000
_NST = M // _B6
_SETS = [(0, 1, 2, True), (3, 4, 2, False)]


def _motif_body(g_ref, w1_ref, b1_ref, w2r_ref, b2_ref,
                rs_ref, cs_ref, ps_ref, out_ref, acc):
    i = pl.program_id(0)

    @pl.when(i == 0)
    def _():
        acc[...] = jnp.zeros_like(acc)

    w1 = w1_ref[...]
    wa, wb, wc = w1[0:32], w1[32:64], w1[64:96]
    b1 = b1_ref[...]
    w2r = w2r_ref[...]
    b2 = b2_ref[...]
    for si, (qa, qb, qc, pos) in enumerate(_SETS):
        ga, gb, gc = g_ref[qa], g_ref[qb], g_ref[qc]
        for t in range(3):
            cols = slice(t * 32, t * 32 + 32)
            pre = (jnp.dot(ga[:, cols], wa, preferred_element_type=jnp.float32)
                   + jnp.dot(gb[:, cols], wb, preferred_element_type=jnp.float32)
                   + jnp.dot(gc[:, cols], wc, preferred_element_type=jnp.float32)
                   + b1)
            h = jnp.maximum(pre, 0.0)
            z = jnp.sum(h * w2r, axis=-1, keepdims=True) + b2
            sg = 1.0 / (1.0 + jnp.exp(-z))
            pp = jnp.clip(sg, 1e-6, 1.0 - 1e-6)
            val = -jnp.log(pp) if pos else -jnp.log(1.0 - pp)
            sidx = t * 2 + si
            acc[sidx, :] = acc[sidx, :] + jnp.sum(val)

    @pl.when(i == _NST - 1)
    def _():
        rsv = rs_ref[...]
        csv = cs_ref[...]
        psv = ps_ref[...]
        l1 = jnp.sum(-jnp.log(psv / (csv - psv) + EPS)) / float(CL_B)
        l2 = jnp.sum(-jnp.log(psv / (rsv - psv) + EPS)) / float(CL_B)
        m = acc[...][:, 0:1]
        mot = jnp.sum(m[0:6]) / float(M)
        out_ref[...] = jnp.full((1, 128), (l1 + l2) * 0.5 + mot, jnp.float32)


_tc_motif = pl.pallas_call(
    _motif_body,
    grid=(_NST,),
    in_specs=[
        pl.BlockSpec((5, _B6, 128), lambda i: (0, i, 0)),
        pl.BlockSpec((96, 64), lambda i: (0, 0)),
        pl.BlockSpec((1, 64), lambda i: (0, 0)),
        pl.BlockSpec((1, 64), lambda i: (0, 0)),
        pl.BlockSpec((1, 1), lambda i: (0, 0)),
        pl.BlockSpec((_NB, _R2), lambda i: (0, 0)),
        pl.BlockSpec((_NB, _R2), lambda i: (0, 0)),
        pl.BlockSpec((_NB, _R2), lambda i: (0, 0)),
    ],
    out_specs=pl.BlockSpec((1, 128), lambda i: (0, 0)),
    out_shape=jax.ShapeDtypeStruct((1, 128), jnp.float32),
    scratch_shapes=[pltpu.VMEM((8, 128), jnp.float32)],
)


def kernel(x, edge_index, motif, neg_motif, rm_feat0, rm_feat1, rm_feat_free,
           W1, b1, W2, b2, Ws0, Ws1, Ws2, bias0, bias1, bias2,
           mc_W1, mc_b1, mc_W2, mc_b2):
    src_flat = edge_index[0].astype(jnp.int32)
    dst_flat = edge_index[1].astype(jnp.int32)
    idxs = [motif[0].astype(jnp.int32), motif[1].astype(jnp.int32),
            motif[2].astype(jnp.int32), neg_motif[0].astype(jnp.int32),
            neg_motif[1].astype(jnp.int32)]

    ones128 = jnp.ones((ECH, 128), jnp.float32)
    zeros128 = jnp.zeros((128, 128), jnp.float32)

    p0, p1, ptab = _tc_prep(rm_feat0, rm_feat1, rm_feat_free)
    degp = _sc_deg(dst_flat, ones128, zeros128)
    dega, degb = degp[0], degp[1]
    xs = _tc_xs(dega, degb, x)
    g1 = _segsum_full(src_flat, dst_flat, xs, zeros128)
    hs = _tc_layer1(g1[0], g1[1], dega, degb, W1, b1.reshape(1, 128))
    g2 = _segsum_cl(src_flat, dst_flat, hs, zeros128)
    G = _sc_gather(ptab, idxs)
    rs, cs, ps = _tc_cl(g2[0], g2[1], dega, degb, W2, b2.reshape(1, 192),
                        p0, p1, rm_feat_free, Ws0, Ws1, Ws2,
                        bias0.reshape(1, 64), bias1.reshape(1, 64),
                        bias2.reshape(1, 64))
    loss = _tc_motif(G, mc_W1, mc_b1.reshape(1, 64),
                     mc_W2.reshape(1, 64), mc_b2.reshape(1, 1),
                     rs, cs, ps)[0, 0]
    return (p0, p1, rm_feat_free, loss)


# speedup vs baseline: 1.1444x; 1.1444x over previous
import functools
import math

import jax
import jax.numpy as jnp
from jax import lax
from jax.experimental import pallas as pl
from jax.experimental.pallas import tpu as pltpu
from jax.experimental.pallas import tpu_sc as plsc

EPS = 1e-5
N = 10000
E = 320000
D_IN = 128
D_HID = 128
D_EMB = 192
D_FACT = 32
D_EMBEDS = 64
M = 100000
TEMP = 0.2
CL_B = 4096

NC = 2
NS = 16
NW = NC * NS

EPW = E // NW
ECH = 80
NECH = EPW // ECH
MCH = 80
NMCH = M // MCH

def _mesh():
    return plsc.VectorSubcoreMesh(core_axis_name="c", subcore_axis_name="s")


_RCH = 80
_NRCH = N // _RCH


def _chunk_loop(nchunks, fn):
    s = lax.axis_index("s")

    def body(u, carry):
        j = s + u * NS

        @pl.when(j < nchunks)
        def _():
            fn(j)
        return carry

    lax.fori_loop(0, (nchunks + NS - 1) // NS, body, 0)


@functools.lru_cache(maxsize=None)
def _build_deg():
    @functools.partial(
        pl.kernel, mesh=_mesh(),
        out_type=jax.ShapeDtypeStruct((NC, N, 128), jnp.float32),
        scratch_types=[
            pltpu.VMEM((ECH,), jnp.int32),
            pltpu.VMEM((ECH,), jnp.int32),
            pltpu.VMEM((ECH, 128), jnp.float32),
            pltpu.VMEM((_RCH, 128), jnp.float32),
            pltpu.VMEM_SHARED((N, 128), jnp.float32),
            pltpu.SemaphoreType.DMA,
            pltpu.SemaphoreType.DMA,
            pltpu.SemaphoreType.DMA,
            pltpu.SemaphoreType.DMA,
        ],
    )
    def k(dst_hbm, ones_h, zeros_h, out_hbm, didx0, didx1, ones_v, zbuf,
          table, ds0, ds1, ss0, ss1):
        c = lax.axis_index("c")
        s = lax.axis_index("s")
        wid = s * NC + c
        pltpu.sync_copy(zeros_h.at[pl.ds(0, _RCH)], zbuf)
        _chunk_loop(_NRCH,
                    lambda j: pltpu.sync_copy(zbuf, table.at[pl.ds(j * _RCH, _RCH)]))
        pltpu.sync_copy(ones_h, ones_v)
        plsc.subcore_barrier()
        base = wid * EPW
        didxs = (didx0, didx1)
        dsem = (ds0, ds1)
        ssem = (ss0, ss1)
        pltpu.async_copy(dst_hbm.at[pl.ds(base, ECH)], didx0, ds0)

        def dbody(j, carry):
            for b in range(2):
                nb = 1 - b

                @pl.when(lax.rem(j, 2) == b)
                def _():
                    pltpu.make_async_copy(
                        dst_hbm.at[pl.ds(base, ECH)], didxs[b], dsem[b]).wait()
                    pltpu.async_copy(ones_v, table.at[didxs[b]], ssem[b],
                                     add=True)

                    @pl.when(j + 1 < NECH)
                    def _():
                        @pl.when(j > 0)
                        def _():
                            pltpu.make_async_copy(
                                ones_v, table.at[didxs[nb]], ssem[nb]).wait()
                        pltpu.async_copy(
                            dst_hbm.at[pl.ds(base + (j + 1) * ECH, ECH)],
                            didxs[nb], dsem[nb])
            return carry

        lax.fori_loop(0, NECH, dbody, 0)
        pltpu.make_async_copy(ones_v, table.at[didx0], ss0).wait()
        pltpu.make_async_copy(ones_v, table.at[didx1], ss1).wait()
        plsc.subcore_barrier()

        def out_chunk(j):
            rows = pl.ds(j * _RCH, _RCH)
            pltpu.sync_copy(table.at[rows], zbuf)
            pltpu.sync_copy(zbuf, out_hbm.at[c, rows])

        _chunk_loop(_NRCH, out_chunk)

    return k


def _sc_deg(dst_flat, ones_hbm, zeros_hbm):
    return _build_deg()(dst_flat, ones_hbm, zeros_hbm)


@functools.lru_cache(maxsize=None)
def _build_gather():
    nu = (NMCH + NW - 1) // NW

    @functools.partial(
        pl.kernel, mesh=_mesh(),
        out_type=jax.ShapeDtypeStruct((5, M, 128), jnp.float32),
        scratch_types=[
            pltpu.VMEM((nu * MCH,), jnp.int32),
            pltpu.VMEM((MCH, 128), jnp.float32),
            pltpu.VMEM((MCH, 128), jnp.float32),
            pltpu.SemaphoreType.DMA,
            pltpu.SemaphoreType.DMA,
            pltpu.SemaphoreType.DMA,
            pltpu.SemaphoreType.DMA,
            pltpu.SemaphoreType.DMA,
        ],
    )
    def k(ptab, i0, i1, i2, i3, i4, gout_hbm,
          idxall, rows0, rows1, gs0, gs1, ws0, ws1, isem):
        c = lax.axis_index("c")
        s = lax.axis_index("s")
        wid = s * NC + c
        rows = (rows0, rows1)
        gsem = (gs0, gs1)
        wsem = (ws0, ws1)
        for q, idx_hbm in enumerate((i0, i1, i2, i3, i4)):
            def ibody(u, carry):
                j = wid + u * NW

                @pl.when(j < NMCH)
                def _():
                    pltpu.async_copy(idx_hbm.at[pl.ds(j * MCH, MCH)],
                                     idxall.at[pl.ds(u * MCH, MCH)], isem)
                return carry

            lax.fori_loop(0, nu, ibody, 0)

            def dbody(u, carry):
                j = wid + u * NW

                @pl.when(j < NMCH)
                def _():
                    pltpu.make_async_copy(
                        idx_hbm.at[pl.ds(wid * MCH, MCH)],
                        idxall.at[pl.ds(0, MCH)], isem).wait()
                return carry

            lax.fori_loop(0, nu, dbody, 0)

            pltpu.async_copy(ptab.at[idxall.at[pl.ds(0, MCH)]], rows0, gs0)

            def body(u, carry):
                j = wid + u * NW
                jn = j + NW

                @pl.when(j < NMCH)
                def _():
                    for b in range(2):
                        nb = 1 - b

                        @pl.when(lax.rem(u, 2) == b)
                        def _():
                            pltpu.make_async_copy(
                                ptab.at[idxall.at[pl.ds(0, MCH)]], rows[b],
                                gsem[b]).wait()
                            pltpu.async_copy(
                                rows[b], gout_hbm.at[q, pl.ds(j * MCH, MCH)],
                                wsem[b])

                            @pl.when(jn < NMCH)
                            def _():
                                @pl.when(u > 0)
                                def _():
                                    pltpu.make_async_copy(
                                        rows[nb],
                                        gout_hbm.at[q, pl.ds(jn * MCH, MCH)],
                                        wsem[nb]).wait()
                                pltpu.async_copy(
                                    ptab.at[idxall.at[pl.ds((u + 1) * MCH, MCH)]],
                                    rows[nb], gsem[nb])
                return carry

            lax.fori_loop(0, nu, body, 0)
            pltpu.make_async_copy(rows0, gout_hbm.at[q, pl.ds(wid * MCH, MCH)],
                                  ws0).wait()
            pltpu.make_async_copy(rows1, gout_hbm.at[q, pl.ds(wid * MCH, MCH)],
                                  ws1).wait()

    return k


def _sc_gather(ptab, idxs):
    return _build_gather()(ptab, *idxs)


@functools.lru_cache(maxsize=None)
def _make_segsum(out_n, cpy):
    @functools.partial(
        pl.kernel, mesh=_mesh(),
        out_type=jax.ShapeDtypeStruct((NC, out_n, 128), jnp.float32),
        scratch_types=[
            pltpu.VMEM((EPW,), jnp.int32),
            pltpu.VMEM((ECH,), jnp.int32),
            pltpu.VMEM((ECH,), jnp.int32),
            pltpu.VMEM((ECH, 128), jnp.float32),
            pltpu.VMEM((ECH, 128), jnp.float32),
            pltpu.VMEM((128, 128), jnp.float32),
            pltpu.VMEM_SHARED((N, 128), jnp.float32),
            pltpu.SemaphoreType.DMA,
            pltpu.SemaphoreType.DMA,
            pltpu.SemaphoreType.DMA,
            pltpu.SemaphoreType.DMA,
            pltpu.SemaphoreType.DMA,
            pltpu.SemaphoreType.DMA,
        ],
    )
    def k(src_hbm, dst_hbm, vals_hbm, zeros_h, out_hbm,
          sidx, didx0, didx1, rows0, rows1, zbuf, table,
          sem0, sem1, ds0, ds1, ss0, ss1):
        c = lax.axis_index("c")
        s = lax.axis_index("s")
        wid = s * NC + c
        base = wid * EPW
        pltpu.sync_copy(src_hbm.at[pl.ds(base, EPW)], sidx)
        pltpu.sync_copy(zeros_h, zbuf)
        _chunk_loop(_NRCH,
                    lambda j: pltpu.sync_copy(zbuf.at[pl.ds(0, _RCH)],
                                              table.at[pl.ds(j * _RCH, _RCH)]))
        plsc.subcore_barrier()

        didxs = (didx0, didx1)
        rows = (rows0, rows1)
        gsem = (sem0, sem1)
        dsem = (ds0, ds1)
        pltpu.async_copy(vals_hbm.at[sidx.at[pl.ds(0, ECH)]], rows0, sem0)
        pltpu.async_copy(dst_hbm.at[pl.ds(base, ECH)], didx0, ds0)

        def body(j, carry):
            @pl.when(j + 1 < NECH)
            def _():
                nxt = sidx.at[pl.ds((j + 1) * ECH, ECH)]
                ndst = dst_hbm.at[pl.ds(base + (j + 1) * ECH, ECH)]

                @pl.when(lax.rem(j, 2) == 0)
                def _():
                    pltpu.async_copy(vals_hbm.at[nxt], rows1, sem1)
                    pltpu.async_copy(ndst, didx1, ds1)

                @pl.when(lax.rem(j, 2) == 1)
                def _():
                    pltpu.async_copy(vals_hbm.at[nxt], rows0, sem0)
                    pltpu.async_copy(ndst, didx0, ds0)

            for b in range(2):
                @pl.when(lax.rem(j, 2) == b)
                def _():
                    pltpu.make_async_copy(
                        dst_hbm.at[pl.ds(base, ECH)], didxs[b], dsem[b]).wait()
                    pltpu.make_async_copy(
                        vals_hbm.at[sidx.at[pl.ds(0, ECH)]], rows[b],
                        gsem[b]).wait()
                    pltpu.sync_copy(rows[b], table.at[didxs[b]], add=True)
            return carry

        lax.fori_loop(0, NECH, body, 0)
        plsc.subcore_barrier()

        def out_chunk(j):
            rows = pl.ds(j * cpy, cpy)
            pltpu.sync_copy(table.at[rows], zbuf.at[pl.ds(0, cpy)])
            pltpu.sync_copy(zbuf.at[pl.ds(0, cpy)], out_hbm.at[c, rows])

        _chunk_loop(out_n // cpy, out_chunk)

    return k


def _segsum_full(src_flat, dst_flat, vals, zeros_hbm):
    return _make_segsum(N, 80)(src_flat, dst_flat, vals, zeros_hbm)


def _segsum_cl(src_flat, dst_flat, vals, zeros_hbm):
    return _make_segsum(CL_B, 64)(src_flat, dst_flat, vals, zeros_hbm)


_R1 = 1000


def _prep_body(r0, r1, r2, p0_ref, p1_ref, pt_ref):
    ps = []
    for (r_ref, p_ref, kk) in ((r0, p0_ref, 0.5), (r1, p1_ref, -0.3)):
        f = r_ref[...]
        radius = 1.0 / math.sqrt(abs(kk))
        nrm = jnp.sqrt(jnp.sum(f * f, axis=-1, keepdims=True)) + EPS
        p = f / nrm * (0.45 * radius)
        p_ref[...] = p
        ps.append(p)
    ps.append(r2[...])
    ps.append(jnp.zeros((ps[0].shape[0], 32), jnp.float32))
    pt_ref[...] = jnp.concatenate(ps, axis=-1)


_tc_prep = pl.pallas_call(
    _prep_body,
    grid=(N // _R1,),
    in_specs=[
        pl.BlockSpec((_R1, 32), lambda i: (i, 0)),
        pl.BlockSpec((_R1, 32), lambda i: (i, 0)),
        pl.BlockSpec((_R1, 32), lambda i: (i, 0)),
    ],
    out_specs=[
        pl.BlockSpec((_R1, 32), lambda i: (i, 0)),
        pl.BlockSpec((_R1, 32), lambda i: (i, 0)),
        pl.BlockSpec((_R1, 128), lambda i: (i, 0)),
    ],
    out_shape=[
        jax.ShapeDtypeStruct((N, 32), jnp.float32),
        jax.ShapeDtypeStruct((N, 32), jnp.float32),
        jax.ShapeDtypeStruct((N, 128), jnp.float32),
    ],
)


def _dis_of(dega, degb):
    deg = dega[:, 0:1] + degb[:, 0:1]
    return 1.0 / jnp.sqrt(jnp.maximum(deg, 1.0))


def _xs_body(dega, degb, x_ref, xs_ref):
    xs_ref[...] = x_ref[...] * _dis_of(dega[...], degb[...])


_tc_xs = pl.pallas_call(
    _xs_body,
    grid=(N // _R1,),
    in_specs=[
        pl.BlockSpec((_R1, 128), lambda i: (i, 0)),
        pl.BlockSpec((_R1, 128), lambda i: (i, 0)),
        pl.BlockSpec((_R1, 128), lambda i: (i, 0)),
    ],
    out_specs=pl.BlockSpec((_R1, 128), lambda i: (i, 0)),
    out_shape=jax.ShapeDtypeStruct((N, 128), jnp.float32),
)


def _layer1_body(g1a, g1b, dega, degb, w1, b1, out_ref):
    dis = _dis_of(dega[...], degb[...])
    g = (g1a[...] + g1b[...]) * dis
    h = jnp.dot(g, w1[...], preferred_element_type=jnp.float32) + b1[...]
    out_ref[...] = jnp.maximum(h, 0.0) * dis


_tc_layer1 = pl.pallas_call(
    _layer1_body,
    grid=(N // _R1,),
    in_specs=[
        pl.BlockSpec((_R1, 128), lambda i: (i, 0)),
        pl.BlockSpec((_R1, 128), lambda i: (i, 0)),
        pl.BlockSpec((_R1, 128), lambda i: (i, 0)),
        pl.BlockSpec((_R1, 128), lambda i: (i, 0)),
        pl.BlockSpec((128, 128), lambda i: (0, 0)),
        pl.BlockSpec((1, 128), lambda i: (0, 0)),
    ],
    out_specs=pl.BlockSpec((_R1, 128), lambda i: (i, 0)),
    out_shape=jax.ShapeDtypeStruct((N, 128), jnp.float32),
)


_R2 = 512
_NB = CL_B // _R2


def _lap_feats(p, w, b, kk):
    pw = lax.dot_general(p, w, (((1,), (1,)), ((), ())),
                         preferred_element_type=jnp.float32)
    if kk == 0.0:
        dist = pw
    else:
        xx = jnp.sum(p * p, axis=-1, keepdims=True)
        ww = jnp.sum(w * w, axis=-1)[None, :]
        div = xx - 2.0 * pw + ww
        dist = jnp.log((1.0 + kk * xx) / (div + EPS))
    return jnp.exp((D_FACT - 1) * dist / 2.0) * jnp.cos(dist + b)


def _cl_body(g2a, g2b, dega, degb, w2, bias2v, p0, p1, p2,
             ws0, ws1, ws2, bs0, bs1, bs2,
             rs_out, cs_out, ps_out, h4s, laps, rs, cs, ps):
    i = pl.program_id(0)
    j = pl.program_id(1)

    @pl.when(j == 0)
    def _():
        dis = _dis_of(dega[...], degb[...])
        g = (g2a[...] + g2b[...]) * dis
        h4s[...] = (jnp.dot(g, w2[...], preferred_element_type=jnp.float32)
                    + bias2v[...])

    @pl.when(i == 0)
    def _():
        laps[pl.ds(j * _R2, _R2), :] = jnp.concatenate(
            [_lap_feats(p0[...], ws0[...], bs0[...], 0.5),
             _lap_feats(p1[...], ws1[...], bs1[...], -0.3),
             _lap_feats(p2[...], ws2[...], bs2[...], 0.0)], axis=-1)

    hb = h4s[...]
    lb = laps[pl.ds(j * _R2, _R2), :]
    n1 = jnp.sqrt(jnp.sum(hb * hb, axis=-1, keepdims=True))
    n2 = jnp.sqrt(jnp.sum(lb * lb, axis=-1))[None, :]
    d = lax.dot_general(hb, lb, (((1,), (1,)), ((), ())),
                        preferred_element_type=jnp.float32)
    s = jnp.exp(d / (n1 * n2 + EPS) / TEMP)
    rowv = jnp.sum(s, axis=1)[None, :]
    colv = jnp.sum(s, axis=0)[None, :]

    @pl.when(j == 0)
    def _():
        rs[pl.ds(i, 1), :] = rowv

    @pl.when(j != 0)
    def _():
        rs[pl.ds(i, 1), :] += rowv

    @pl.when(i == 0)
    def _():
        cs[pl.ds(j, 1), :] = colv

    @pl.when(i != 0)
    def _():
        cs[pl.ds(j, 1), :] += colv

    @pl.when(i == j)
    def _():
        rr = lax.broadcasted_iota(jnp.int32, (_R2, _R2), 0)
        cc = lax.broadcasted_iota(jnp.int32, (_R2, _R2), 1)
        diag = jnp.sum(jnp.where(rr == cc, s, 0.0), axis=1)[None, :]
        ps[pl.ds(i, 1), :] = diag

    @pl.when((i == _NB - 1) & (j == _NB - 1))
    def _():
        rs_out[...] = rs[...]
        cs_out[...] = cs[...]
        ps_out[...] = ps[...]


_tc_cl = pl.pallas_call(
    _cl_body,
    grid=(_NB, _NB),
    in_specs=[
        pl.BlockSpec((_R2, 128), lambda i, j: (i, 0)),
        pl.BlockSpec((_R2, 128), lambda i, j: (i, 0)),
        pl.BlockSpec((_R2, 128), lambda i, j: (i, 0)),
        pl.BlockSpec((_R2, 128), lambda i, j: (i, 0)),
        pl.BlockSpec((128, 192), lambda i, j: (0, 0)),
        pl.BlockSpec((1, 192), lambda i, j: (0, 0)),
        pl.BlockSpec((_R2, 32), lambda i, j: (j, 0)),
        pl.BlockSpec((_R2, 32), lambda i, j: (j, 0)),
        pl.BlockSpec((_R2, 32), lambda i, j: (j, 0)),
        pl.BlockSpec((64, 32), lambda i, j: (0, 0)),
        pl.BlockSpec((64, 32), lambda i, j: (0, 0)),
        pl.BlockSpec((64, 32), lambda i, j: (0, 0)),
        pl.BlockSpec((1, 64), lambda i, j: (0, 0)),
        pl.BlockSpec((1, 64), lambda i, j: (0, 0)),
        pl.BlockSpec((1, 64), lambda i, j: (0, 0)),
    ],
    out_specs=[
        pl.BlockSpec((_NB, _R2), lambda i, j: (0, 0)),
        pl.BlockSpec((_NB, _R2), lambda i, j: (0, 0)),
        pl.BlockSpec((_NB, _R2), lambda i, j: (0, 0)),
    ],
    out_shape=[
        jax.ShapeDtypeStruct((_NB, _R2), jnp.float32),
        jax.ShapeDtypeStruct((_NB, _R2), jnp.float32),
        jax.ShapeDtypeStruct((_NB, _R2), jnp.float32),
    ],
    scratch_shapes=[
        pltpu.VMEM((_R2, 192), jnp.float32),
        pltpu.VMEM((CL_B, 192), jnp.float32),
        pltpu.VMEM((_NB, _R2), jnp.float32),
        pltpu.VMEM((_NB, _R2), jnp.float32),
        pltpu.VMEM((_NB, _R2), jnp.float32),
    ],
)


_B6 = 2000
_NST = M // _B6
_SETS = [(0, 1, 2, True), (3, 4, 2, False)]


def _motif_body(g_ref, w1_ref, b1_ref, w2r_ref, b2_ref,
                rs_ref, cs_ref, ps_ref, out_ref, acc):
    i = pl.program_id(0)

    @pl.when(i == 0)
    def _():
        acc[...] = jnp.zeros_like(acc)

    w1 = w1_ref[...]
    wa, wb, wc = w1[0:32], w1[32:64], w1[64:96]
    b1 = b1_ref[...]
    w2r = w2r_ref[...]
    b2 = b2_ref[...]
    for si, (qa, qb, qc, pos) in enumerate(_SETS):
        ga, gb, gc = g_ref[qa], g_ref[qb], g_ref[qc]
        for t in range(3):
            cols = slice(t * 32, t * 32 + 32)
            pre = (jnp.dot(ga[:, cols], wa, preferred_element_type=jnp.float32)
                   + jnp.dot(gb[:, cols], wb, preferred_element_type=jnp.float32)
                   + jnp.dot(gc[:, cols], wc, preferred_element_type=jnp.float32)
                   + b1)
            h = jnp.maximum(pre, 0.0)
            z = jnp.sum(h * w2r, axis=-1, keepdims=True) + b2
            sg = 1.0 / (1.0 + jnp.exp(-z))
            pp = jnp.clip(sg, 1e-6, 1.0 - 1e-6)
            val = -jnp.log(pp) if pos else -jnp.log(1.0 - pp)
            sidx = t * 2 + si
            acc[sidx, :] = acc[sidx, :] + jnp.sum(val)

    @pl.when(i == _NST - 1)
    def _():
        rsv = rs_ref[...]
        csv = cs_ref[...]
        psv = ps_ref[...]
        l1 = jnp.sum(-jnp.log(psv / (csv - psv) + EPS)) / float(CL_B)
        l2 = jnp.sum(-jnp.log(psv / (rsv - psv) + EPS)) / float(CL_B)
        m = acc[...][:, 0:1]
        mot = jnp.sum(m[0:6]) / float(M)
        out_ref[...] = jnp.full((1, 128), (l1 + l2) * 0.5 + mot, jnp.float32)


_tc_motif = pl.pallas_call(
    _motif_body,
    grid=(_NST,),
    in_specs=[
        pl.BlockSpec((5, _B6, 128), lambda i: (0, i, 0)),
        pl.BlockSpec((96, 64), lambda i: (0, 0)),
        pl.BlockSpec((1, 64), lambda i: (0, 0)),
        pl.BlockSpec((1, 64), lambda i: (0, 0)),
        pl.BlockSpec((1, 1), lambda i: (0, 0)),
        pl.BlockSpec((_NB, _R2), lambda i: (0, 0)),
        pl.BlockSpec((_NB, _R2), lambda i: (0, 0)),
        pl.BlockSpec((_NB, _R2), lambda i: (0, 0)),
    ],
    out_specs=pl.BlockSpec((1, 128), lambda i: (0, 0)),
    out_shape=jax.ShapeDtypeStruct((1, 128), jnp.float32),
    scratch_shapes=[pltpu.VMEM((8, 128), jnp.float32)],
)


def kernel(x, edge_index, motif, neg_motif, rm_feat0, rm_feat1, rm_feat_free,
           W1, b1, W2, b2, Ws0, Ws1, Ws2, bias0, bias1, bias2,
           mc_W1, mc_b1, mc_W2, mc_b2):
    src_flat = edge_index[0].astype(jnp.int32)
    dst_flat = edge_index[1].astype(jnp.int32)
    idxs = [motif[0].astype(jnp.int32), motif[1].astype(jnp.int32),
            motif[2].astype(jnp.int32), neg_motif[0].astype(jnp.int32),
            neg_motif[1].astype(jnp.int32)]

    ones128 = jnp.ones((ECH, 128), jnp.float32)
    zeros128 = jnp.zeros((128, 128), jnp.float32)

    p0, p1, ptab = _tc_prep(rm_feat0, rm_feat1, rm_feat_free)
    degp = _sc_deg(dst_flat, ones128, zeros128)
    dega, degb = degp[0], degp[1]
    xs = _tc_xs(dega, degb, x)
    g1 = _segsum_full(src_flat, dst_flat, xs, zeros128)
    hs = _tc_layer1(g1[0], g1[1], dega, degb, W1, b1.reshape(1, 128))
    g2 = _segsum_cl(src_flat, dst_flat, hs, zeros128)
    G = _sc_gather(ptab, idxs)
    rs, cs, ps = _tc_cl(g2[0], g2[1], dega, degb, W2, b2.reshape(1, 192),
                        p0, p1, rm_feat_free, Ws0, Ws1, Ws2,
                        bias0.reshape(1, 64), bias1.reshape(1, 64),
                        bias2.reshape(1, 64))
    loss = _tc_motif(G, mc_W1, mc_b1.reshape(1, 64),
                     mc_W2.reshape(1, 64), mc_b2.reshape(1, 1),
                     rs, cs, ps)[0, 0]
    return (p0, p1, rm_feat_free, loss)


# speedup vs baseline: 1.2411x; 1.0845x over previous
import functools
import math

import jax
import jax.numpy as jnp
from jax import lax
from jax.experimental import pallas as pl
from jax.experimental.pallas import tpu as pltpu
from jax.experimental.pallas import tpu_sc as plsc

EPS = 1e-5
N = 10000
E = 320000
D_IN = 128
D_HID = 128
D_EMB = 192
D_FACT = 32
D_EMBEDS = 64
M = 100000
TEMP = 0.2
CL_B = 4096

NC = 2
NS = 16
NW = NC * NS

EPW = E // NW
ECH = 80
NECH = EPW // ECH
MCH = 80
NMCH = M // MCH

def _mesh():
    return plsc.VectorSubcoreMesh(core_axis_name="c", subcore_axis_name="s")


_RCH = 80
_NRCH = N // _RCH


def _chunk_loop(nchunks, fn):
    s = lax.axis_index("s")

    def body(u, carry):
        j = s + u * NS

        @pl.when(j < nchunks)
        def _():
            fn(j)
        return carry

    lax.fori_loop(0, (nchunks + NS - 1) // NS, body, 0)


@functools.lru_cache(maxsize=None)
def _build_deg():
    @functools.partial(
        pl.kernel, mesh=_mesh(),
        out_type=jax.ShapeDtypeStruct((NC, N, 128), jnp.float32),
        scratch_types=[
            pltpu.VMEM((ECH,), jnp.int32),
            pltpu.VMEM((ECH,), jnp.int32),
            pltpu.VMEM((ECH, 128), jnp.float32),
            pltpu.VMEM((_RCH, 128), jnp.float32),
            pltpu.VMEM_SHARED((N, 128), jnp.float32),
            pltpu.SemaphoreType.DMA,
            pltpu.SemaphoreType.DMA,
            pltpu.SemaphoreType.DMA,
            pltpu.SemaphoreType.DMA,
        ],
    )
    def k(dst_hbm, ones_h, zeros_h, out_hbm, didx0, didx1, ones_v, zbuf,
          table, ds0, ds1, ss0, ss1):
        c = lax.axis_index("c")
        s = lax.axis_index("s")
        wid = s * NC + c
        pltpu.sync_copy(zeros_h.at[pl.ds(0, _RCH)], zbuf)
        _chunk_loop(_NRCH,
                    lambda j: pltpu.sync_copy(zbuf, table.at[pl.ds(j * _RCH, _RCH)]))
        pltpu.sync_copy(ones_h, ones_v)
        plsc.subcore_barrier()
        base = wid * EPW
        didxs = (didx0, didx1)
        dsem = (ds0, ds1)
        ssem = (ss0, ss1)
        pltpu.async_copy(dst_hbm.at[pl.ds(base, ECH)], didx0, ds0)

        def dbody(j, carry):
            for b in range(2):
                nb = 1 - b

                @pl.when(lax.rem(j, 2) == b)
                def _():
                    pltpu.make_async_copy(
                        dst_hbm.at[pl.ds(base, ECH)], didxs[b], dsem[b]).wait()
                    pltpu.async_copy(ones_v, table.at[didxs[b]], ssem[b],
                                     add=True)

                    @pl.when(j + 1 < NECH)
                    def _():
                        @pl.when(j > 0)
                        def _():
                            pltpu.make_async_copy(
                                ones_v, table.at[didxs[nb]], ssem[nb]).wait()
                        pltpu.async_copy(
                            dst_hbm.at[pl.ds(base + (j + 1) * ECH, ECH)],
                            didxs[nb], dsem[nb])
            return carry

        lax.fori_loop(0, NECH, dbody, 0)
        pltpu.make_async_copy(ones_v, table.at[didx0], ss0).wait()
        pltpu.make_async_copy(ones_v, table.at[didx1], ss1).wait()
        plsc.subcore_barrier()

        def out_chunk(j):
            rows = pl.ds(j * _RCH, _RCH)
            pltpu.sync_copy(table.at[rows], zbuf)
            pltpu.sync_copy(zbuf, out_hbm.at[c, rows])

        _chunk_loop(_NRCH, out_chunk)

    return k


def _sc_deg(dst_flat, ones_hbm, zeros_hbm):
    return _build_deg()(dst_flat, ones_hbm, zeros_hbm)


@functools.lru_cache(maxsize=None)
def _build_gather():
    nu = (NMCH + NW - 1) // NW

    @functools.partial(
        pl.kernel, mesh=_mesh(),
        out_type=jax.ShapeDtypeStruct((5, M, 128), jnp.float32),
        scratch_types=[
            pltpu.VMEM((nu * MCH,), jnp.int32),
            pltpu.VMEM((MCH, 128), jnp.float32),
            pltpu.VMEM((MCH, 128), jnp.float32),
            pltpu.VMEM((MCH, 128), jnp.float32),
            pltpu.SemaphoreType.DMA,
            pltpu.SemaphoreType.DMA,
            pltpu.SemaphoreType.DMA,
            pltpu.SemaphoreType.DMA,
            pltpu.SemaphoreType.DMA,
            pltpu.SemaphoreType.DMA,
            pltpu.SemaphoreType.DMA,
        ],
    )
    def k(ptab, i0, i1, i2, i3, i4, gout_hbm,
          idxall, rows0, rows1, rows2, gs0, gs1, gs2, ws0, ws1, ws2, isem):
        c = lax.axis_index("c")
        s = lax.axis_index("s")
        wid = s * NC + c
        rows = (rows0, rows1, rows2)
        gsem = (gs0, gs1, gs2)
        wsem = (ws0, ws1, ws2)
        for q, idx_hbm in enumerate((i0, i1, i2, i3, i4)):
            def ibody(u, carry):
                j = wid + u * NW

                @pl.when(j < NMCH)
                def _():
                    pltpu.async_copy(idx_hbm.at[pl.ds(j * MCH, MCH)],
                                     idxall.at[pl.ds(u * MCH, MCH)], isem)
                return carry

            lax.fori_loop(0, nu, ibody, 0)

            def dbody(u, carry):
                j = wid + u * NW

                @pl.when(j < NMCH)
                def _():
                    pltpu.make_async_copy(
                        idx_hbm.at[pl.ds(wid * MCH, MCH)],
                        idxall.at[pl.ds(0, MCH)], isem).wait()
                return carry

            lax.fori_loop(0, nu, dbody, 0)

            pltpu.async_copy(ptab.at[idxall.at[pl.ds(0, MCH)]], rows0, gs0)
            pltpu.async_copy(ptab.at[idxall.at[pl.ds(MCH, MCH)]], rows1, gs1)

            def body(u, carry):
                j = wid + u * NW

                @pl.when(j < NMCH)
                def _():
                    for b in range(3):
                        nb2 = (b + 2) % 3

                        @pl.when(lax.rem(u, 3) == b)
                        def _():
                            pltpu.make_async_copy(
                                ptab.at[idxall.at[pl.ds(0, MCH)]], rows[b],
                                gsem[b]).wait()
                            pltpu.async_copy(
                                rows[b], gout_hbm.at[q, pl.ds(j * MCH, MCH)],
                                wsem[b])

                            @pl.when(j + 2 * NW < NMCH)
                            def _():
                                @pl.when(u > 0)
                                def _():
                                    pltpu.make_async_copy(
                                        rows[nb2],
                                        gout_hbm.at[q, pl.ds(j * MCH, MCH)],
                                        wsem[nb2]).wait()
                                pltpu.async_copy(
                                    ptab.at[idxall.at[pl.ds((u + 2) * MCH, MCH)]],
                                    rows[nb2], gsem[nb2])
                return carry

            lax.fori_loop(0, nu, body, 0)
            for b in range(3):
                pltpu.make_async_copy(
                    rows[b], gout_hbm.at[q, pl.ds(wid * MCH, MCH)],
                    wsem[b]).wait()

    return k


def _sc_gather(ptab, idxs):
    return _build_gather()(ptab, *idxs)


@functools.lru_cache(maxsize=None)
def _make_segsum(out_n, cpy):
    @functools.partial(
        pl.kernel, mesh=_mesh(),
        out_type=jax.ShapeDtypeStruct((NC, out_n, 128), jnp.float32),
        scratch_types=[
            pltpu.VMEM((EPW,), jnp.int32),
            pltpu.VMEM((ECH,), jnp.int32),
            pltpu.VMEM((ECH,), jnp.int32),
            pltpu.VMEM((ECH, 128), jnp.float32),
            pltpu.VMEM((ECH, 128), jnp.float32),
            pltpu.VMEM((128, 128), jnp.float32),
            pltpu.VMEM_SHARED((N, 128), jnp.float32),
            pltpu.SemaphoreType.DMA,
            pltpu.SemaphoreType.DMA,
            pltpu.SemaphoreType.DMA,
            pltpu.SemaphoreType.DMA,
            pltpu.SemaphoreType.DMA,
            pltpu.SemaphoreType.DMA,
        ],
    )
    def k(src_hbm, dst_hbm, vals_hbm, zeros_h, out_hbm,
          sidx, didx0, didx1, rows0, rows1, zbuf, table,
          sem0, sem1, ds0, ds1, ss0, ss1):
        c = lax.axis_index("c")
        s = lax.axis_index("s")
        wid = s * NC + c
        base = wid * EPW
        pltpu.sync_copy(src_hbm.at[pl.ds(base, EPW)], sidx)
        pltpu.sync_copy(zeros_h, zbuf)
        _chunk_loop(_NRCH,
                    lambda j: pltpu.sync_copy(zbuf.at[pl.ds(0, _RCH)],
                                              table.at[pl.ds(j * _RCH, _RCH)]))
        plsc.subcore_barrier()

        didxs = (didx0, didx1)
        rows = (rows0, rows1)
        gsem = (sem0, sem1)
        dsem = (ds0, ds1)
        pltpu.async_copy(vals_hbm.at[sidx.at[pl.ds(0, ECH)]], rows0, sem0)
        pltpu.async_copy(dst_hbm.at[pl.ds(base, ECH)], didx0, ds0)

        def body(j, carry):
            @pl.when(j + 1 < NECH)
            def _():
                nxt = sidx.at[pl.ds((j + 1) * ECH, ECH)]
                ndst = dst_hbm.at[pl.ds(base + (j + 1) * ECH, ECH)]

                @pl.when(lax.rem(j, 2) == 0)
                def _():
                    pltpu.async_copy(vals_hbm.at[nxt], rows1, sem1)
                    pltpu.async_copy(ndst, didx1, ds1)

                @pl.when(lax.rem(j, 2) == 1)
                def _():
                    pltpu.async_copy(vals_hbm.at[nxt], rows0, sem0)
                    pltpu.async_copy(ndst, didx0, ds0)

            for b in range(2):
                @pl.when(lax.rem(j, 2) == b)
                def _():
                    pltpu.make_async_copy(
                        dst_hbm.at[pl.ds(base, ECH)], didxs[b], dsem[b]).wait()
                    pltpu.make_async_copy(
                        vals_hbm.at[sidx.at[pl.ds(0, ECH)]], rows[b],
                        gsem[b]).wait()
                    pltpu.sync_copy(rows[b], table.at[didxs[b]], add=True)
            return carry

        lax.fori_loop(0, NECH, body, 0)
        plsc.subcore_barrier()

        def out_chunk(j):
            rows = pl.ds(j * cpy, cpy)
            pltpu.sync_copy(table.at[rows], zbuf.at[pl.ds(0, cpy)])
            pltpu.sync_copy(zbuf.at[pl.ds(0, cpy)], out_hbm.at[c, rows])

        _chunk_loop(out_n // cpy, out_chunk)

    return k


def _segsum_full(src_flat, dst_flat, vals, zeros_hbm):
    return _make_segsum(N, 80)(src_flat, dst_flat, vals, zeros_hbm)


def _segsum_cl(src_flat, dst_flat, vals, zeros_hbm):
    return _make_segsum(CL_B, 64)(src_flat, dst_flat, vals, zeros_hbm)


_R1 = 1000


def _prep_body(r0, r1, r2, p0_ref, p1_ref, pt_ref):
    ps = []
    for (r_ref, p_ref, kk) in ((r0, p0_ref, 0.5), (r1, p1_ref, -0.3)):
        f = r_ref[...]
        radius = 1.0 / math.sqrt(abs(kk))
        nrm = jnp.sqrt(jnp.sum(f * f, axis=-1, keepdims=True)) + EPS
        p = f / nrm * (0.45 * radius)
        p_ref[...] = p
        ps.append(p)
    ps.append(r2[...])
    ps.append(jnp.zeros((ps[0].shape[0], 32), jnp.float32))
    pt_ref[...] = jnp.concatenate(ps, axis=-1)


_tc_prep = pl.pallas_call(
    _prep_body,
    grid=(N // _R1,),
    in_specs=[
        pl.BlockSpec((_R1, 32), lambda i: (i, 0)),
        pl.BlockSpec((_R1, 32), lambda i: (i, 0)),
        pl.BlockSpec((_R1, 32), lambda i: (i, 0)),
    ],
    out_specs=[
        pl.BlockSpec((_R1, 32), lambda i: (i, 0)),
        pl.BlockSpec((_R1, 32), lambda i: (i, 0)),
        pl.BlockSpec((_R1, 128), lambda i: (i, 0)),
    ],
    out_shape=[
        jax.ShapeDtypeStruct((N, 32), jnp.float32),
        jax.ShapeDtypeStruct((N, 32), jnp.float32),
        jax.ShapeDtypeStruct((N, 128), jnp.float32),
    ],
)


def _dis_of(dega, degb):
    deg = dega[:, 0:1] + degb[:, 0:1]
    return 1.0 / jnp.sqrt(jnp.maximum(deg, 1.0))


def _xs_body(dega, degb, x_ref, xs_ref):
    xs_ref[...] = x_ref[...] * _dis_of(dega[...], degb[...])


_tc_xs = pl.pallas_call(
    _xs_body,
    grid=(N // _R1,),
    in_specs=[
        pl.BlockSpec((_R1, 128), lambda i: (i, 0)),
        pl.BlockSpec((_R1, 128), lambda i: (i, 0)),
        pl.BlockSpec((_R1, 128), lambda i: (i, 0)),
    ],
    out_specs=pl.BlockSpec((_R1, 128), lambda i: (i, 0)),
    out_shape=jax.ShapeDtypeStruct((N, 128), jnp.float32),
)


def _layer1_body(g1a, g1b, dega, degb, w1, b1, out_ref):
    dis = _dis_of(dega[...], degb[...])
    g = (g1a[...] + g1b[...]) * dis
    h = jnp.dot(g, w1[...], preferred_element_type=jnp.float32) + b1[...]
    out_ref[...] = jnp.maximum(h, 0.0) * dis


_tc_layer1 = pl.pallas_call(
    _layer1_body,
    grid=(N // _R1,),
    in_specs=[
        pl.BlockSpec((_R1, 128), lambda i: (i, 0)),
        pl.BlockSpec((_R1, 128), lambda i: (i, 0)),
        pl.BlockSpec((_R1, 128), lambda i: (i, 0)),
        pl.BlockSpec((_R1, 128), lambda i: (i, 0)),
        pl.BlockSpec((128, 128), lambda i: (0, 0)),
        pl.BlockSpec((1, 128), lambda i: (0, 0)),
    ],
    out_specs=pl.BlockSpec((_R1, 128), lambda i: (i, 0)),
    out_shape=jax.ShapeDtypeStruct((N, 128), jnp.float32),
)


_R2 = 512
_NB = CL_B // _R2


def _lap_feats(p, w, b, kk):
    pw = lax.dot_general(p, w, (((1,), (1,)), ((), ())),
                         preferred_element_type=jnp.float32)
    if kk == 0.0:
        dist = pw
    else:
        xx = jnp.sum(p * p, axis=-1, keepdims=True)
        ww = jnp.sum(w * w, axis=-1)[None, :]
        div = xx - 2.0 * pw + ww
        dist = jnp.log((1.0 + kk * xx) / (div + EPS))
    return jnp.exp((D_FACT - 1) * dist / 2.0) * jnp.cos(dist + b)


def _cl_body(g2a, g2b, dega, degb, w2, bias2v, p0, p1, p2,
             ws0, ws1, ws2, bs0, bs1, bs2,
             rs_out, cs_out, ps_out, h4s, laps, rs, cs, ps):
    i = pl.program_id(0)
    j = pl.program_id(1)

    @pl.when(j == 0)
    def _():
        dis = _dis_of(dega[...], degb[...])
        g = (g2a[...] + g2b[...]) * dis
        h4s[...] = (jnp.dot(g, w2[...], preferred_element_type=jnp.float32)
                    + bias2v[...])

    @pl.when(i == 0)
    def _():
        laps[pl.ds(j * _R2, _R2), :] = jnp.concatenate(
            [_lap_feats(p0[...], ws0[...], bs0[...], 0.5),
             _lap_feats(p1[...], ws1[...], bs1[...], -0.3),
             _lap_feats(p2[...], ws2[...], bs2[...], 0.0)], axis=-1)

    hb = h4s[...]
    lb = laps[pl.ds(j * _R2, _R2), :]
    n1 = jnp.sqrt(jnp.sum(hb * hb, axis=-1, keepdims=True))
    n2 = jnp.sqrt(jnp.sum(lb * lb, axis=-1))[None, :]
    d = lax.dot_general(hb, lb, (((1,), (1,)), ((), ())),
                        preferred_element_type=jnp.float32)
    s = jnp.exp(d / (n1 * n2 + EPS) / TEMP)
    rowv = jnp.sum(s, axis=1)[None, :]
    colv = jnp.sum(s, axis=0)[None, :]

    @pl.when(j == 0)
    def _():
        rs[pl.ds(i, 1), :] = rowv

    @pl.when(j != 0)
    def _():
        rs[pl.ds(i, 1), :] += rowv

    @pl.when(i == 0)
    def _():
        cs[pl.ds(j, 1), :] = colv

    @pl.when(i != 0)
    def _():
        cs[pl.ds(j, 1), :] += colv

    @pl.when(i == j)
    def _():
        rr = lax.broadcasted_iota(jnp.int32, (_R2, _R2), 0)
        cc = lax.broadcasted_iota(jnp.int32, (_R2, _R2), 1)
        diag = jnp.sum(jnp.where(rr == cc, s, 0.0), axis=1)[None, :]
        ps[pl.ds(i, 1), :] = diag

    @pl.when((i == _NB - 1) & (j == _NB - 1))
    def _():
        rs_out[...] = rs[...]
        cs_out[...] = cs[...]
        ps_out[...] = ps[...]


_tc_cl = pl.pallas_call(
    _cl_body,
    grid=(_NB, _NB),
    in_specs=[
        pl.BlockSpec((_R2, 128), lambda i, j: (i, 0)),
        pl.BlockSpec((_R2, 128), lambda i, j: (i, 0)),
        pl.BlockSpec((_R2, 128), lambda i, j: (i, 0)),
        pl.BlockSpec((_R2, 128), lambda i, j: (i, 0)),
        pl.BlockSpec((128, 192), lambda i, j: (0, 0)),
        pl.BlockSpec((1, 192), lambda i, j: (0, 0)),
        pl.BlockSpec((_R2, 32), lambda i, j: (j, 0)),
        pl.BlockSpec((_R2, 32), lambda i, j: (j, 0)),
        pl.BlockSpec((_R2, 32), lambda i, j: (j, 0)),
        pl.BlockSpec((64, 32), lambda i, j: (0, 0)),
        pl.BlockSpec((64, 32), lambda i, j: (0, 0)),
        pl.BlockSpec((64, 32), lambda i, j: (0, 0)),
        pl.BlockSpec((1, 64), lambda i, j: (0, 0)),
        pl.BlockSpec((1, 64), lambda i, j: (0, 0)),
        pl.BlockSpec((1, 64), lambda i, j: (0, 0)),
    ],
    out_specs=[
        pl.BlockSpec((_NB, _R2), lambda i, j: (0, 0)),
        pl.BlockSpec((_NB, _R2), lambda i, j: (0, 0)),
        pl.BlockSpec((_NB, _R2), lambda i, j: (0, 0)),
    ],
    out_shape=[
        jax.ShapeDtypeStruct((_NB, _R2), jnp.float32),
        jax.ShapeDtypeStruct((_NB, _R2), jnp.float32),
        jax.ShapeDtypeStruct((_NB, _R2), jnp.float32),
    ],
    scratch_shapes=[
        pltpu.VMEM((_R2, 192), jnp.float32),
        pltpu.VMEM((CL_B, 192), jnp.float32),
        pltpu.VMEM((_NB, _R2), jnp.float32),
        pltpu.VMEM((_NB, _R2), jnp.float32),
        pltpu.VMEM((_NB, _R2), jnp.float32),
    ],
)


_B6 = 2000
_NST = M // _B6
_SETS = [(0, 1, 2, True), (3, 4, 2, False)]


def _motif_body(g_ref, w1_ref, b1_ref, w2r_ref, b2_ref,
                rs_ref, cs_ref, ps_ref, out_ref, acc):
    i = pl.program_id(0)

    @pl.when(i == 0)
    def _():
        acc[...] = jnp.zeros_like(acc)

    w1 = w1_ref[...]
    wa, wb, wc = w1[0:32], w1[32:64], w1[64:96]
    b1 = b1_ref[...]
    w2r = w2r_ref[...]
    b2 = b2_ref[...]
    for si, (qa, qb, qc, pos) in enumerate(_SETS):
        ga, gb, gc = g_ref[qa], g_ref[qb], g_ref[qc]
        for t in range(3):
            cols = slice(t * 32, t * 32 + 32)
            pre = (jnp.dot(ga[:, cols], wa, preferred_element_type=jnp.float32)
                   + jnp.dot(gb[:, cols], wb, preferred_element_type=jnp.float32)
                   + jnp.dot(gc[:, cols], wc, preferred_element_type=jnp.float32)
                   + b1)
            h = jnp.maximum(pre, 0.0)
            z = jnp.sum(h * w2r, axis=-1, keepdims=True) + b2
            sg = 1.0 / (1.0 + jnp.exp(-z))
            pp = jnp.clip(sg, 1e-6, 1.0 - 1e-6)
            val = -jnp.log(pp) if pos else -jnp.log(1.0 - pp)
            sidx = t * 2 + si
            acc[sidx, :] = acc[sidx, :] + jnp.sum(val)

    @pl.when(i == _NST - 1)
    def _():
        rsv = rs_ref[...]
        csv = cs_ref[...]
        psv = ps_ref[...]
        l1 = jnp.sum(-jnp.log(psv / (csv - psv) + EPS)) / float(CL_B)
        l2 = jnp.sum(-jnp.log(psv / (rsv - psv) + EPS)) / float(CL_B)
        m = acc[...][:, 0:1]
        mot = jnp.sum(m[0:6]) / float(M)
        out_ref[...] = jnp.full((1, 128), (l1 + l2) * 0.5 + mot, jnp.float32)


_tc_motif = pl.pallas_call(
    _motif_body,
    grid=(_NST,),
    in_specs=[
        pl.BlockSpec((5, _B6, 128), lambda i: (0, i, 0)),
        pl.BlockSpec((96, 64), lambda i: (0, 0)),
        pl.BlockSpec((1, 64), lambda i: (0, 0)),
        pl.BlockSpec((1, 64), lambda i: (0, 0)),
        pl.BlockSpec((1, 1), lambda i: (0, 0)),
        pl.BlockSpec((_NB, _R2), lambda i: (0, 0)),
        pl.BlockSpec((_NB, _R2), lambda i: (0, 0)),
        pl.BlockSpec((_NB, _R2), lambda i: (0, 0)),
    ],
    out_specs=pl.BlockSpec((1, 128), lambda i: (0, 0)),
    out_shape=jax.ShapeDtypeStruct((1, 128), jnp.float32),
    scratch_shapes=[pltpu.VMEM((8, 128), jnp.float32)],
)


def kernel(x, edge_index, motif, neg_motif, rm_feat0, rm_feat1, rm_feat_free,
           W1, b1, W2, b2, Ws0, Ws1, Ws2, bias0, bias1, bias2,
           mc_W1, mc_b1, mc_W2, mc_b2):
    src_flat = edge_index[0].astype(jnp.int32)
    dst_flat = edge_index[1].astype(jnp.int32)
    idxs = [motif[0].astype(jnp.int32), motif[1].astype(jnp.int32),
            motif[2].astype(jnp.int32), neg_motif[0].astype(jnp.int32),
            neg_motif[1].astype(jnp.int32)]

    ones128 = jnp.ones((ECH, 128), jnp.float32)
    zeros128 = jnp.zeros((128, 128), jnp.float32)

    p0, p1, ptab = _tc_prep(rm_feat0, rm_feat1, rm_feat_free)
    degp = _sc_deg(dst_flat, ones128, zeros128)
    dega, degb = degp[0], degp[1]
    xs = _tc_xs(dega, degb, x)
    g1 = _segsum_full(src_flat, dst_flat, xs, zeros128)
    hs = _tc_layer1(g1[0], g1[1], dega, degb, W1, b1.reshape(1, 128))
    g2 = _segsum_cl(src_flat, dst_flat, hs, zeros128)
    G = _sc_gather(ptab, idxs)
    rs, cs, ps = _tc_cl(g2[0], g2[1], dega, degb, W2, b2.reshape(1, 192),
                        p0, p1, rm_feat_free, Ws0, Ws1, Ws2,
                        bias0.reshape(1, 64), bias1.reshape(1, 64),
                        bias2.reshape(1, 64))
    loss = _tc_motif(G, mc_W1, mc_b1.reshape(1, 64),
                     mc_W2.reshape(1, 64), mc_b2.reshape(1, 1),
                     rs, cs, ps)[0, 0]
    return (p0, p1, rm_feat_free, loss)


# speedup vs baseline: 1.3009x; 1.0482x over previous
import functools
import math

import jax
import jax.numpy as jnp
from jax import lax
from jax.experimental import pallas as pl
from jax.experimental.pallas import tpu as pltpu
from jax.experimental.pallas import tpu_sc as plsc

EPS = 1e-5
N = 10000
E = 320000
D_IN = 128
D_HID = 128
D_EMB = 192
D_FACT = 32
D_EMBEDS = 64
M = 100000
TEMP = 0.2
CL_B = 4096

NC = 2
NS = 16
NW = NC * NS

EPW = E // NW
ECH = 80
NECH = EPW // ECH
MCH = 80
NMCH = M // MCH

def _mesh():
    return plsc.VectorSubcoreMesh(core_axis_name="c", subcore_axis_name="s")


_RCH = 80
_NRCH = N // _RCH


def _chunk_loop(nchunks, fn):
    s = lax.axis_index("s")

    def body(u, carry):
        j = s + u * NS

        @pl.when(j < nchunks)
        def _():
            fn(j)
        return carry

    lax.fori_loop(0, (nchunks + NS - 1) // NS, body, 0)


@functools.lru_cache(maxsize=None)
def _build_deg():
    @functools.partial(
        pl.kernel, mesh=_mesh(),
        out_type=jax.ShapeDtypeStruct((NC, N, 128), jnp.float32),
        scratch_types=[
            pltpu.VMEM((ECH,), jnp.int32),
            pltpu.VMEM((ECH,), jnp.int32),
            pltpu.VMEM((ECH, 128), jnp.float32),
            pltpu.VMEM((_RCH, 128), jnp.float32),
            pltpu.VMEM_SHARED((N, 128), jnp.float32),
            pltpu.SemaphoreType.DMA,
            pltpu.SemaphoreType.DMA,
            pltpu.SemaphoreType.DMA,
            pltpu.SemaphoreType.DMA,
        ],
    )
    def k(dst_hbm, ones_h, zeros_h, out_hbm, didx0, didx1, ones_v, zbuf,
          table, ds0, ds1, ss0, ss1):
        c = lax.axis_index("c")
        s = lax.axis_index("s")
        wid = s * NC + c
        pltpu.sync_copy(zeros_h.at[pl.ds(0, _RCH)], zbuf)
        _chunk_loop(_NRCH,
                    lambda j: pltpu.sync_copy(zbuf, table.at[pl.ds(j * _RCH, _RCH)]))
        pltpu.sync_copy(ones_h, ones_v)
        plsc.subcore_barrier()
        base = wid * EPW
        didxs = (didx0, didx1)
        dsem = (ds0, ds1)
        ssem = (ss0, ss1)
        pltpu.async_copy(dst_hbm.at[pl.ds(base, ECH)], didx0, ds0)

        def dbody(j, carry):
            for b in range(2):
                nb = 1 - b

                @pl.when(lax.rem(j, 2) == b)
                def _():
                    pltpu.make_async_copy(
                        dst_hbm.at[pl.ds(base, ECH)], didxs[b], dsem[b]).wait()
                    pltpu.async_copy(ones_v, table.at[didxs[b]], ssem[b],
                                     add=True)

                    @pl.when(j + 1 < NECH)
                    def _():
                        @pl.when(j > 0)
                        def _():
                            pltpu.make_async_copy(
                                ones_v, table.at[didxs[nb]], ssem[nb]).wait()
                        pltpu.async_copy(
                            dst_hbm.at[pl.ds(base + (j + 1) * ECH, ECH)],
                            didxs[nb], dsem[nb])
            return carry

        lax.fori_loop(0, NECH, dbody, 0)
        pltpu.make_async_copy(ones_v, table.at[didx0], ss0).wait()
        pltpu.make_async_copy(ones_v, table.at[didx1], ss1).wait()
        plsc.subcore_barrier()

        def out_chunk(j):
            rows = pl.ds(j * _RCH, _RCH)
            pltpu.sync_copy(table.at[rows], zbuf)
            pltpu.sync_copy(zbuf, out_hbm.at[c, rows])

        _chunk_loop(_NRCH, out_chunk)

    return k


def _sc_deg(dst_flat, ones_hbm, zeros_hbm):
    return _build_deg()(dst_flat, ones_hbm, zeros_hbm)


@functools.lru_cache(maxsize=None)
def _build_gather():
    nu = (NMCH + NW - 1) // NW

    @functools.partial(
        pl.kernel, mesh=_mesh(),
        out_type=jax.ShapeDtypeStruct((5, M, 128), jnp.float32),
        scratch_types=[
            pltpu.VMEM((nu * MCH,), jnp.int32),
            pltpu.VMEM((MCH, 128), jnp.float32),
            pltpu.VMEM((MCH, 128), jnp.float32),
            pltpu.VMEM((MCH, 128), jnp.float32),
            pltpu.SemaphoreType.DMA,
            pltpu.SemaphoreType.DMA,
            pltpu.SemaphoreType.DMA,
            pltpu.SemaphoreType.DMA,
            pltpu.SemaphoreType.DMA,
            pltpu.SemaphoreType.DMA,
            pltpu.SemaphoreType.DMA,
        ],
    )
    def k(ptab, i0, i1, i2, i3, i4, gout_hbm,
          idxall, rows0, rows1, rows2, gs0, gs1, gs2, ws0, ws1, ws2, isem):
        c = lax.axis_index("c")
        s = lax.axis_index("s")
        wid = s * NC + c
        rows = (rows0, rows1, rows2)
        gsem = (gs0, gs1, gs2)
        wsem = (ws0, ws1, ws2)
        for q, idx_hbm in enumerate((i0, i1, i2, i3, i4)):
            def ibody(u, carry):
                j = wid + u * NW

                @pl.when(j < NMCH)
                def _():
                    pltpu.async_copy(idx_hbm.at[pl.ds(j * MCH, MCH)],
                                     idxall.at[pl.ds(u * MCH, MCH)], isem)
                return carry

            lax.fori_loop(0, nu, ibody, 0)

            def dbody(u, carry):
                j = wid + u * NW

                @pl.when(j < NMCH)
                def _():
                    pltpu.make_async_copy(
                        idx_hbm.at[pl.ds(wid * MCH, MCH)],
                        idxall.at[pl.ds(0, MCH)], isem).wait()
                return carry

            lax.fori_loop(0, nu, dbody, 0)

            pltpu.async_copy(ptab.at[idxall.at[pl.ds(0, MCH)]], rows0, gs0)
            pltpu.async_copy(ptab.at[idxall.at[pl.ds(MCH, MCH)]], rows1, gs1)

            def body(u, carry):
                j = wid + u * NW

                @pl.when(j < NMCH)
                def _():
                    for b in range(3):
                        nb2 = (b + 2) % 3

                        @pl.when(lax.rem(u, 3) == b)
                        def _():
                            pltpu.make_async_copy(
                                ptab.at[idxall.at[pl.ds(0, MCH)]], rows[b],
                                gsem[b]).wait()
                            pltpu.async_copy(
                                rows[b], gout_hbm.at[q, pl.ds(j * MCH, MCH)],
                                wsem[b])

                            @pl.when(j + 2 * NW < NMCH)
                            def _():
                                @pl.when(u > 0)
                                def _():
                                    pltpu.make_async_copy(
                                        rows[nb2],
                                        gout_hbm.at[q, pl.ds(j * MCH, MCH)],
                                        wsem[nb2]).wait()
                                pltpu.async_copy(
                                    ptab.at[idxall.at[pl.ds((u + 2) * MCH, MCH)]],
                                    rows[nb2], gsem[nb2])
                return carry

            lax.fori_loop(0, nu, body, 0)
            for b in range(3):
                pltpu.make_async_copy(
                    rows[b], gout_hbm.at[q, pl.ds(wid * MCH, MCH)],
                    wsem[b]).wait()

    return k


def _sc_gather(ptab, idxs):
    return _build_gather()(ptab, *idxs)


@functools.lru_cache(maxsize=None)
def _make_segsum(out_n, cpy):
    @functools.partial(
        pl.kernel, mesh=_mesh(),
        out_type=jax.ShapeDtypeStruct((NC, out_n, 128), jnp.float32),
        scratch_types=[
            pltpu.VMEM((EPW,), jnp.int32),
            pltpu.VMEM((ECH,), jnp.int32),
            pltpu.VMEM((ECH,), jnp.int32),
            pltpu.VMEM((ECH,), jnp.int32),
            pltpu.VMEM((ECH, 128), jnp.float32),
            pltpu.VMEM((ECH, 128), jnp.float32),
            pltpu.VMEM((ECH, 128), jnp.float32),
            pltpu.VMEM_SHARED((N, 128), jnp.float32),
            pltpu.SemaphoreType.DMA,
            pltpu.SemaphoreType.DMA,
            pltpu.SemaphoreType.DMA,
            pltpu.SemaphoreType.DMA,
            pltpu.SemaphoreType.DMA,
            pltpu.SemaphoreType.DMA,
        ],
    )
    def k(src_hbm, dst_hbm, vals_hbm, zeros_h, out_hbm,
          sidx, didx0, didx1, didx2, rows0, rows1, rows2, table,
          sem0, sem1, sem2, ds0, ds1, ds2):
        c = lax.axis_index("c")
        s = lax.axis_index("s")
        wid = s * NC + c
        base = wid * EPW
        pltpu.sync_copy(src_hbm.at[pl.ds(base, EPW)], sidx)
        pltpu.sync_copy(zeros_h.at[pl.ds(0, _RCH)], rows0)
        _chunk_loop(_NRCH,
                    lambda j: pltpu.sync_copy(rows0,
                                              table.at[pl.ds(j * _RCH, _RCH)]))
        plsc.subcore_barrier()

        didxs = (didx0, didx1, didx2)
        rows = (rows0, rows1, rows2)
        gsem = (sem0, sem1, sem2)
        dsem = (ds0, ds1, ds2)
        pltpu.async_copy(vals_hbm.at[sidx.at[pl.ds(0, ECH)]], rows0, sem0)
        pltpu.async_copy(dst_hbm.at[pl.ds(base, ECH)], didx0, ds0)
        pltpu.async_copy(vals_hbm.at[sidx.at[pl.ds(ECH, ECH)]], rows1, sem1)
        pltpu.async_copy(dst_hbm.at[pl.ds(base + ECH, ECH)], didx1, ds1)

        def body(j, carry):
            for b in range(3):
                nb2 = (b + 2) % 3

                @pl.when(lax.rem(j, 3) == b)
                def _():
                    @pl.when(j + 2 < NECH)
                    def _():
                        pltpu.async_copy(
                            vals_hbm.at[sidx.at[pl.ds((j + 2) * ECH, ECH)]],
                            rows[nb2], gsem[nb2])
                        pltpu.async_copy(
                            dst_hbm.at[pl.ds(base + (j + 2) * ECH, ECH)],
                            didxs[nb2], dsem[nb2])

                    pltpu.make_async_copy(
                        dst_hbm.at[pl.ds(base, ECH)], didxs[b], dsem[b]).wait()
                    pltpu.make_async_copy(
                        vals_hbm.at[sidx.at[pl.ds(0, ECH)]], rows[b],
                        gsem[b]).wait()
                    pltpu.sync_copy(rows[b], table.at[didxs[b]], add=True)
            return carry

        lax.fori_loop(0, NECH, body, 0)
        plsc.subcore_barrier()

        def out_chunk(j):
            rsl = pl.ds(j * cpy, cpy)
            pltpu.sync_copy(table.at[rsl], rows0.at[pl.ds(0, cpy)])
            pltpu.sync_copy(rows0.at[pl.ds(0, cpy)], out_hbm.at[c, rsl])

        _chunk_loop(out_n // cpy, out_chunk)

    return k


def _segsum_full(src_flat, dst_flat, vals, zeros_hbm):
    return _make_segsum(N, 80)(src_flat, dst_flat, vals, zeros_hbm)


def _segsum_cl(src_flat, dst_flat, vals, zeros_hbm):
    return _make_segsum(CL_B, 64)(src_flat, dst_flat, vals, zeros_hbm)


_R1 = 1000


def _prep_body(r0, r1, r2, p0_ref, p1_ref, pt_ref):
    ps = []
    for (r_ref, p_ref, kk) in ((r0, p0_ref, 0.5), (r1, p1_ref, -0.3)):
        f = r_ref[...]
        radius = 1.0 / math.sqrt(abs(kk))
        nrm = jnp.sqrt(jnp.sum(f * f, axis=-1, keepdims=True)) + EPS
        p = f / nrm * (0.45 * radius)
        p_ref[...] = p
        ps.append(p)
    ps.append(r2[...])
    ps.append(jnp.zeros((ps[0].shape[0], 32), jnp.float32))
    pt_ref[...] = jnp.concatenate(ps, axis=-1)


_tc_prep = pl.pallas_call(
    _prep_body,
    grid=(N // _R1,),
    in_specs=[
        pl.BlockSpec((_R1, 32), lambda i: (i, 0)),
        pl.BlockSpec((_R1, 32), lambda i: (i, 0)),
        pl.BlockSpec((_R1, 32), lambda i: (i, 0)),
    ],
    out_specs=[
        pl.BlockSpec((_R1, 32), lambda i: (i, 0)),
        pl.BlockSpec((_R1, 32), lambda i: (i, 0)),
        pl.BlockSpec((_R1, 128), lambda i: (i, 0)),
    ],
    out_shape=[
        jax.ShapeDtypeStruct((N, 32), jnp.float32),
        jax.ShapeDtypeStruct((N, 32), jnp.float32),
        jax.ShapeDtypeStruct((N, 128), jnp.float32),
    ],
)


def _dis_of(dega, degb):
    deg = dega[:, 0:1] + degb[:, 0:1]
    return 1.0 / jnp.sqrt(jnp.maximum(deg, 1.0))


def _xs_body(dega, degb, x_ref, xs_ref):
    xs_ref[...] = x_ref[...] * _dis_of(dega[...], degb[...])


_tc_xs = pl.pallas_call(
    _xs_body,
    grid=(N // _R1,),
    in_specs=[
        pl.BlockSpec((_R1, 128), lambda i: (i, 0)),
        pl.BlockSpec((_R1, 128), lambda i: (i, 0)),
        pl.BlockSpec((_R1, 128), lambda i: (i, 0)),
    ],
    out_specs=pl.BlockSpec((_R1, 128), lambda i: (i, 0)),
    out_shape=jax.ShapeDtypeStruct((N, 128), jnp.float32),
)


def _layer1_body(g1a, g1b, dega, degb, w1, b1, out_ref):
    dis = _dis_of(dega[...], degb[...])
    g = (g1a[...] + g1b[...]) * dis
    h = jnp.dot(g, w1[...], preferred_element_type=jnp.float32) + b1[...]
    out_ref[...] = jnp.maximum(h, 0.0) * dis


_tc_layer1 = pl.pallas_call(
    _layer1_body,
    grid=(N // _R1,),
    in_specs=[
        pl.BlockSpec((_R1, 128), lambda i: (i, 0)),
        pl.BlockSpec((_R1, 128), lambda i: (i, 0)),
        pl.BlockSpec((_R1, 128), lambda i: (i, 0)),
        pl.BlockSpec((_R1, 128), lambda i: (i, 0)),
        pl.BlockSpec((128, 128), lambda i: (0, 0)),
        pl.BlockSpec((1, 128), lambda i: (0, 0)),
    ],
    out_specs=pl.BlockSpec((_R1, 128), lambda i: (i, 0)),
    out_shape=jax.ShapeDtypeStruct((N, 128), jnp.float32),
)


_R2 = 512
_NB = CL_B // _R2


def _lap_feats(p, w, b, kk):
    pw = lax.dot_general(p, w, (((1,), (1,)), ((), ())),
                         preferred_element_type=jnp.float32)
    if kk == 0.0:
        dist = pw
    else:
        xx = jnp.sum(p * p, axis=-1, keepdims=True)
        ww = jnp.sum(w * w, axis=-1)[None, :]
        div = xx - 2.0 * pw + ww
        dist = jnp.log((1.0 + kk * xx) / (div + EPS))
    return jnp.exp((D_FACT - 1) * dist / 2.0) * jnp.cos(dist + b)


def _cl_body(g2a, g2b, dega, degb, w2, bias2v, p0, p1, p2,
             ws0, ws1, ws2, bs0, bs1, bs2,
             rs_out, cs_out, ps_out, h4s, laps, rs, cs, ps):
    i = pl.program_id(0)
    j = pl.program_id(1)

    @pl.when(j == 0)
    def _():
        dis = _dis_of(dega[...], degb[...])
        g = (g2a[...] + g2b[...]) * dis
        h4s[...] = (jnp.dot(g, w2[...], preferred_element_type=jnp.float32)
                    + bias2v[...])

    @pl.when(i == 0)
    def _():
        laps[pl.ds(j * _R2, _R2), :] = jnp.concatenate(
            [_lap_feats(p0[...], ws0[...], bs0[...], 0.5),
             _lap_feats(p1[...], ws1[...], bs1[...], -0.3),
             _lap_feats(p2[...], ws2[...], bs2[...], 0.0)], axis=-1)

    hb = h4s[...]
    lb = laps[pl.ds(j * _R2, _R2), :]
    n1 = jnp.sqrt(jnp.sum(hb * hb, axis=-1, keepdims=True))
    n2 = jnp.sqrt(jnp.sum(lb * lb, axis=-1))[None, :]
    d = lax.dot_general(hb, lb, (((1,), (1,)), ((), ())),
                        preferred_element_type=jnp.float32)
    s = jnp.exp(d / (n1 * n2 + EPS) / TEMP)
    rowv = jnp.sum(s, axis=1)[None, :]
    colv = jnp.sum(s, axis=0)[None, :]

    @pl.when(j == 0)
    def _():
        rs[pl.ds(i, 1), :] = rowv

    @pl.when(j != 0)
    def _():
        rs[pl.ds(i, 1), :] += rowv

    @pl.when(i == 0)
    def _():
        cs[pl.ds(j, 1), :] = colv

    @pl.when(i != 0)
    def _():
        cs[pl.ds(j, 1), :] += colv

    @pl.when(i == j)
    def _():
        rr = lax.broadcasted_iota(jnp.int32, (_R2, _R2), 0)
        cc = lax.broadcasted_iota(jnp.int32, (_R2, _R2), 1)
        diag = jnp.sum(jnp.where(rr == cc, s, 0.0), axis=1)[None, :]
        ps[pl.ds(i, 1), :] = diag

    @pl.when((i == _NB - 1) & (j == _NB - 1))
    def _():
        rs_out[...] = rs[...]
        cs_out[...] = cs[...]
        ps_out[...] = ps[...]


_tc_cl = pl.pallas_call(
    _cl_body,
    grid=(_NB, _NB),
    in_specs=[
        pl.BlockSpec((_R2, 128), lambda i, j: (i, 0)),
        pl.BlockSpec((_R2, 128), lambda i, j: (i, 0)),
        pl.BlockSpec((_R2, 128), lambda i, j: (i, 0)),
        pl.BlockSpec((_R2, 128), lambda i, j: (i, 0)),
        pl.BlockSpec((128, 192), lambda i, j: (0, 0)),
        pl.BlockSpec((1, 192), lambda i, j: (0, 0)),
        pl.BlockSpec((_R2, 32), lambda i, j: (j, 0)),
        pl.BlockSpec((_R2, 32), lambda i, j: (j, 0)),
        pl.BlockSpec((_R2, 32), lambda i, j: (j, 0)),
        pl.BlockSpec((64, 32), lambda i, j: (0, 0)),
        pl.BlockSpec((64, 32), lambda i, j: (0, 0)),
        pl.BlockSpec((64, 32), lambda i, j: (0, 0)),
        pl.BlockSpec((1, 64), lambda i, j: (0, 0)),
        pl.BlockSpec((1, 64), lambda i, j: (0, 0)),
        pl.BlockSpec((1, 64), lambda i, j: (0, 0)),
    ],
    out_specs=[
        pl.BlockSpec((_NB, _R2), lambda i, j: (0, 0)),
        pl.BlockSpec((_NB, _R2), lambda i, j: (0, 0)),
        pl.BlockSpec((_NB, _R2), lambda i, j: (0, 0)),
    ],
    out_shape=[
        jax.ShapeDtypeStruct((_NB, _R2), jnp.float32),
        jax.ShapeDtypeStruct((_NB, _R2), jnp.float32),
        jax.ShapeDtypeStruct((_NB, _R2), jnp.float32),
    ],
    scratch_shapes=[
        pltpu.VMEM((_R2, 192), jnp.float32),
        pltpu.VMEM((CL_B, 192), jnp.float32),
        pltpu.VMEM((_NB, _R2), jnp.float32),
        pltpu.VMEM((_NB, _R2), jnp.float32),
        pltpu.VMEM((_NB, _R2), jnp.float32),
    ],
)


_B6 = 2000
_NST = M // _B6
_SETS = [(0, 1, 2, True), (3, 4, 2, False)]


def _motif_body(g_ref, w1_ref, b1_ref, w2r_ref, b2_ref,
                rs_ref, cs_ref, ps_ref, out_ref, acc):
    i = pl.program_id(0)

    @pl.when(i == 0)
    def _():
        acc[...] = jnp.zeros_like(acc)

    w1 = w1_ref[...]
    wa, wb, wc = w1[0:32], w1[32:64], w1[64:96]
    b1 = b1_ref[...]
    w2r = w2r_ref[...]
    b2 = b2_ref[...]
    for si, (qa, qb, qc, pos) in enumerate(_SETS):
        ga, gb, gc = g_ref[qa], g_ref[qb], g_ref[qc]
        for t in range(3):
            cols = slice(t * 32, t * 32 + 32)
            pre = (jnp.dot(ga[:, cols], wa, preferred_element_type=jnp.float32)
                   + jnp.dot(gb[:, cols], wb, preferred_element_type=jnp.float32)
                   + jnp.dot(gc[:, cols], wc, preferred_element_type=jnp.float32)
                   + b1)
            h = jnp.maximum(pre, 0.0)
            z = jnp.sum(h * w2r, axis=-1, keepdims=True) + b2
            sg = 1.0 / (1.0 + jnp.exp(-z))
            pp = jnp.clip(sg, 1e-6, 1.0 - 1e-6)
            val = -jnp.log(pp) if pos else -jnp.log(1.0 - pp)
            sidx = t * 2 + si
            acc[sidx, :] = acc[sidx, :] + jnp.sum(val)

    @pl.when(i == _NST - 1)
    def _():
        rsv = rs_ref[...]
        csv = cs_ref[...]
        psv = ps_ref[...]
        l1 = jnp.sum(-jnp.log(psv / (csv - psv) + EPS)) / float(CL_B)
        l2 = jnp.sum(-jnp.log(psv / (rsv - psv) + EPS)) / float(CL_B)
        m = acc[...][:, 0:1]
        mot = jnp.sum(m[0:6]) / float(M)
        out_ref[...] = jnp.full((1, 128), (l1 + l2) * 0.5 + mot, jnp.float32)


_tc_motif = pl.pallas_call(
    _motif_body,
    grid=(_NST,),
    in_specs=[
        pl.BlockSpec((5, _B6, 128), lambda i: (0, i, 0)),
        pl.BlockSpec((96, 64), lambda i: (0, 0)),
        pl.BlockSpec((1, 64), lambda i: (0, 0)),
        pl.BlockSpec((1, 64), lambda i: (0, 0)),
        pl.BlockSpec((1, 1), lambda i: (0, 0)),
        pl.BlockSpec((_NB, _R2), lambda i: (0, 0)),
        pl.BlockSpec((_NB, _R2), lambda i: (0, 0)),
        pl.BlockSpec((_NB, _R2), lambda i: (0, 0)),
    ],
    out_specs=pl.BlockSpec((1, 128), lambda i: (0, 0)),
    out_shape=jax.ShapeDtypeStruct((1, 128), jnp.float32),
    scratch_shapes=[pltpu.VMEM((8, 128), jnp.float32)],
)


def kernel(x, edge_index, motif, neg_motif, rm_feat0, rm_feat1, rm_feat_free,
           W1, b1, W2, b2, Ws0, Ws1, Ws2, bias0, bias1, bias2,
           mc_W1, mc_b1, mc_W2, mc_b2):
    src_flat = edge_index[0].astype(jnp.int32)
    dst_flat = edge_index[1].astype(jnp.int32)
    idxs = [motif[0].astype(jnp.int32), motif[1].astype(jnp.int32),
            motif[2].astype(jnp.int32), neg_motif[0].astype(jnp.int32),
            neg_motif[1].astype(jnp.int32)]

    ones128 = jnp.ones((ECH, 128), jnp.float32)
    zeros128 = jnp.zeros((128, 128), jnp.float32)

    p0, p1, ptab = _tc_prep(rm_feat0, rm_feat1, rm_feat_free)
    degp = _sc_deg(dst_flat, ones128, zeros128)
    dega, degb = degp[0], degp[1]
    xs = _tc_xs(dega, degb, x)
    g1 = _segsum_full(src_flat, dst_flat, xs, zeros128)
    hs = _tc_layer1(g1[0], g1[1], dega, degb, W1, b1.reshape(1, 128))
    g2 = _segsum_cl(src_flat, dst_flat, hs, zeros128)
    G = _sc_gather(ptab, idxs)
    rs, cs, ps = _tc_cl(g2[0], g2[1], dega, degb, W2, b2.reshape(1, 192),
                        p0, p1, rm_feat_free, Ws0, Ws1, Ws2,
                        bias0.reshape(1, 64), bias1.reshape(1, 64),
                        bias2.reshape(1, 64))
    loss = _tc_motif(G, mc_W1, mc_b1.reshape(1, 64),
                     mc_W2.reshape(1, 64), mc_b2.reshape(1, 1),
                     rs, cs, ps)[0, 0]
    return (p0, p1, rm_feat_free, loss)


# speedup vs baseline: 1.3039x; 1.0023x over previous
import functools
import math

import jax
import jax.numpy as jnp
from jax import lax
from jax.experimental import pallas as pl
from jax.experimental.pallas import tpu as pltpu
from jax.experimental.pallas import tpu_sc as plsc

EPS = 1e-5
N = 10000
E = 320000
D_IN = 128
D_HID = 128
D_EMB = 192
D_FACT = 32
D_EMBEDS = 64
M = 100000
TEMP = 0.2
CL_B = 4096

NC = 2
NS = 16
NW = NC * NS

EPW = E // NW
ECH = 80
NECH = EPW // ECH
MCH = 80
NMCH = M // MCH

def _mesh():
    return plsc.VectorSubcoreMesh(core_axis_name="c", subcore_axis_name="s")


_RCH = 80
_NRCH = N // _RCH


def _chunk_loop(nchunks, fn):
    s = lax.axis_index("s")

    def body(u, carry):
        j = s + u * NS

        @pl.when(j < nchunks)
        def _():
            fn(j)
        return carry

    lax.fori_loop(0, (nchunks + NS - 1) // NS, body, 0)


@functools.lru_cache(maxsize=None)
def _build_deg():
    @functools.partial(
        pl.kernel, mesh=_mesh(),
        out_type=jax.ShapeDtypeStruct((NC, N, 128), jnp.float32),
        scratch_types=[
            pltpu.VMEM((ECH,), jnp.int32),
            pltpu.VMEM((ECH,), jnp.int32),
            pltpu.VMEM((ECH,), jnp.int32),
            pltpu.VMEM((ECH, 128), jnp.float32),
            pltpu.VMEM((_RCH, 128), jnp.float32),
            pltpu.VMEM_SHARED((N, 128), jnp.float32),
            pltpu.SemaphoreType.DMA,
            pltpu.SemaphoreType.DMA,
            pltpu.SemaphoreType.DMA,
            pltpu.SemaphoreType.DMA,
            pltpu.SemaphoreType.DMA,
            pltpu.SemaphoreType.DMA,
        ],
    )
    def k(dst_hbm, ones_h, zeros_h, out_hbm, didx0, didx1, didx2, ones_v,
          zbuf, table, ds0, ds1, ds2, ss0, ss1, ss2):
        c = lax.axis_index("c")
        s = lax.axis_index("s")
        wid = s * NC + c
        pltpu.sync_copy(zeros_h.at[pl.ds(0, _RCH)], zbuf)
        _chunk_loop(_NRCH,
                    lambda j: pltpu.sync_copy(zbuf, table.at[pl.ds(j * _RCH, _RCH)]))
        pltpu.sync_copy(ones_h, ones_v)
        plsc.subcore_barrier()
        base = wid * EPW
        didxs = (didx0, didx1, didx2)
        dsem = (ds0, ds1, ds2)
        ssem = (ss0, ss1, ss2)
        pltpu.async_copy(dst_hbm.at[pl.ds(base, ECH)], didx0, ds0)
        pltpu.async_copy(dst_hbm.at[pl.ds(base + ECH, ECH)], didx1, ds1)

        def dbody(j, carry):
            for b in range(3):
                nb2 = (b + 2) % 3

                @pl.when(lax.rem(j, 3) == b)
                def _():
                    pltpu.make_async_copy(
                        dst_hbm.at[pl.ds(base, ECH)], didxs[b], dsem[b]).wait()
                    pltpu.async_copy(ones_v, table.at[didxs[b]], ssem[b],
                                     add=True)

                    @pl.when(j + 2 < NECH)
                    def _():
                        @pl.when(j > 0)
                        def _():
                            pltpu.make_async_copy(
                                ones_v, table.at[didxs[nb2]], ssem[nb2]).wait()
                        pltpu.async_copy(
                            dst_hbm.at[pl.ds(base + (j + 2) * ECH, ECH)],
                            didxs[nb2], dsem[nb2])
            return carry

        lax.fori_loop(0, NECH, dbody, 0)
        pltpu.make_async_copy(ones_v, table.at[didx0], ss0).wait()
        pltpu.make_async_copy(ones_v, table.at[didx1], ss1).wait()
        pltpu.make_async_copy(ones_v, table.at[didx2], ss2).wait()
        plsc.subcore_barrier()

        def out_chunk(j):
            rows = pl.ds(j * _RCH, _RCH)
            pltpu.sync_copy(table.at[rows], zbuf)
            pltpu.sync_copy(zbuf, out_hbm.at[c, rows])

        _chunk_loop(_NRCH, out_chunk)

    return k


def _sc_deg(dst_flat, ones_hbm, zeros_hbm):
    return _build_deg()(dst_flat, ones_hbm, zeros_hbm)


@functools.lru_cache(maxsize=None)
def _build_gather():
    nu = (NMCH + NW - 1) // NW

    @functools.partial(
        pl.kernel, mesh=_mesh(),
        out_type=jax.ShapeDtypeStruct((5, M, 128), jnp.float32),
        scratch_types=[
            pltpu.VMEM((nu * MCH,), jnp.int32),
            pltpu.VMEM((MCH, 128), jnp.float32),
            pltpu.VMEM((MCH, 128), jnp.float32),
            pltpu.VMEM((MCH, 128), jnp.float32),
            pltpu.SemaphoreType.DMA,
            pltpu.SemaphoreType.DMA,
            pltpu.SemaphoreType.DMA,
            pltpu.SemaphoreType.DMA,
            pltpu.SemaphoreType.DMA,
            pltpu.SemaphoreType.DMA,
            pltpu.SemaphoreType.DMA,
        ],
    )
    def k(ptab, i0, i1, i2, i3, i4, gout_hbm,
          idxall, rows0, rows1, rows2, gs0, gs1, gs2, ws0, ws1, ws2, isem):
        c = lax.axis_index("c")
        s = lax.axis_index("s")
        wid = s * NC + c
        rows = (rows0, rows1, rows2)
        gsem = (gs0, gs1, gs2)
        wsem = (ws0, ws1, ws2)
        for q, idx_hbm in enumerate((i0, i1, i2, i3, i4)):
            def ibody(u, carry):
                j = wid + u * NW

                @pl.when(j < NMCH)
                def _():
                    pltpu.async_copy(idx_hbm.at[pl.ds(j * MCH, MCH)],
                                     idxall.at[pl.ds(u * MCH, MCH)], isem)
                return carry

            lax.fori_loop(0, nu, ibody, 0)

            def dbody(u, carry):
                j = wid + u * NW

                @pl.when(j < NMCH)
                def _():
                    pltpu.make_async_copy(
                        idx_hbm.at[pl.ds(wid * MCH, MCH)],
                        idxall.at[pl.ds(0, MCH)], isem).wait()
                return carry

            lax.fori_loop(0, nu, dbody, 0)

            pltpu.async_copy(ptab.at[idxall.at[pl.ds(0, MCH)]], rows0, gs0)
            pltpu.async_copy(ptab.at[idxall.at[pl.ds(MCH, MCH)]], rows1, gs1)

            def body(u, carry):
                j = wid + u * NW

                @pl.when(j < NMCH)
                def _():
                    for b in range(3):
                        nb2 = (b + 2) % 3

                        @pl.when(lax.rem(u, 3) == b)
                        def _():
                            pltpu.make_async_copy(
                                ptab.at[idxall.at[pl.ds(0, MCH)]], rows[b],
                                gsem[b]).wait()
                            pltpu.async_copy(
                                rows[b], gout_hbm.at[q, pl.ds(j * MCH, MCH)],
                                wsem[b])

                            @pl.when(j + 2 * NW < NMCH)
                            def _():
                                @pl.when(u > 0)
                                def _():
                                    pltpu.make_async_copy(
                                        rows[nb2],
                                        gout_hbm.at[q, pl.ds(j * MCH, MCH)],
                                        wsem[nb2]).wait()
                                pltpu.async_copy(
                                    ptab.at[idxall.at[pl.ds((u + 2) * MCH, MCH)]],
                                    rows[nb2], gsem[nb2])
                return carry

            lax.fori_loop(0, nu, body, 0)
            for b in range(3):
                pltpu.make_async_copy(
                    rows[b], gout_hbm.at[q, pl.ds(wid * MCH, MCH)],
                    wsem[b]).wait()

    return k


def _sc_gather(ptab, idxs):
    return _build_gather()(ptab, *idxs)


@functools.lru_cache(maxsize=None)
def _make_segsum(out_n, cpy):
    @functools.partial(
        pl.kernel, mesh=_mesh(),
        out_type=jax.ShapeDtypeStruct((NC, out_n, 128), jnp.float32),
        scratch_types=[
            pltpu.VMEM((EPW,), jnp.int32),
            pltpu.VMEM((ECH,), jnp.int32),
            pltpu.VMEM((ECH,), jnp.int32),
            pltpu.VMEM((ECH,), jnp.int32),
            pltpu.VMEM((ECH, 128), jnp.float32),
            pltpu.VMEM((ECH, 128), jnp.float32),
            pltpu.VMEM((ECH, 128), jnp.float32),
            pltpu.VMEM_SHARED((N, 128), jnp.float32),
            pltpu.SemaphoreType.DMA,
            pltpu.SemaphoreType.DMA,
            pltpu.SemaphoreType.DMA,
            pltpu.SemaphoreType.DMA,
            pltpu.SemaphoreType.DMA,
            pltpu.SemaphoreType.DMA,
        ],
    )
    def k(src_hbm, dst_hbm, vals_hbm, zeros_h, out_hbm,
          sidx, didx0, didx1, didx2, rows0, rows1, rows2, table,
          sem0, sem1, sem2, ds0, ds1, ds2):
        c = lax.axis_index("c")
        s = lax.axis_index("s")
        wid = s * NC + c
        base = wid * EPW
        pltpu.sync_copy(src_hbm.at[pl.ds(base, EPW)], sidx)
        pltpu.sync_copy(zeros_h.at[pl.ds(0, _RCH)], rows0)
        _chunk_loop(_NRCH,
                    lambda j: pltpu.sync_copy(rows0,
                                              table.at[pl.ds(j * _RCH, _RCH)]))
        plsc.subcore_barrier()

        didxs = (didx0, didx1, didx2)
        rows = (rows0, rows1, rows2)
        gsem = (sem0, sem1, sem2)
        dsem = (ds0, ds1, ds2)
        pltpu.async_copy(vals_hbm.at[sidx.at[pl.ds(0, ECH)]], rows0, sem0)
        pltpu.async_copy(dst_hbm.at[pl.ds(base, ECH)], didx0, ds0)
        pltpu.async_copy(vals_hbm.at[sidx.at[pl.ds(ECH, ECH)]], rows1, sem1)
        pltpu.async_copy(dst_hbm.at[pl.ds(base + ECH, ECH)], didx1, ds1)

        def body(j, carry):
            for b in range(3):
                nb2 = (b + 2) % 3

                @pl.when(lax.rem(j, 3) == b)
                def _():
                    @pl.when(j + 2 < NECH)
                    def _():
                        pltpu.async_copy(
                            vals_hbm.at[sidx.at[pl.ds((j + 2) * ECH, ECH)]],
                            rows[nb2], gsem[nb2])
                        pltpu.async_copy(
                            dst_hbm.at[pl.ds(base + (j + 2) * ECH, ECH)],
                            didxs[nb2], dsem[nb2])

                    pltpu.make_async_copy(
                        dst_hbm.at[pl.ds(base, ECH)], didxs[b], dsem[b]).wait()
                    pltpu.make_async_copy(
                        vals_hbm.at[sidx.at[pl.ds(0, ECH)]], rows[b],
                        gsem[b]).wait()
                    pltpu.sync_copy(rows[b], table.at[didxs[b]], add=True)
            return carry

        lax.fori_loop(0, NECH, body, 0)
        plsc.subcore_barrier()

        def out_chunk(j):
            rsl = pl.ds(j * cpy, cpy)
            pltpu.sync_copy(table.at[rsl], rows0.at[pl.ds(0, cpy)])
            pltpu.sync_copy(rows0.at[pl.ds(0, cpy)], out_hbm.at[c, rsl])

        _chunk_loop(out_n // cpy, out_chunk)

    return k


def _segsum_full(src_flat, dst_flat, vals, zeros_hbm):
    return _make_segsum(N, 80)(src_flat, dst_flat, vals, zeros_hbm)


def _segsum_cl(src_flat, dst_flat, vals, zeros_hbm):
    return _make_segsum(CL_B, 64)(src_flat, dst_flat, vals, zeros_hbm)


_R1 = 1000


def _prep_body(r0, r1, r2, p0_ref, p1_ref, pt_ref):
    ps = []
    for (r_ref, p_ref, kk) in ((r0, p0_ref, 0.5), (r1, p1_ref, -0.3)):
        f = r_ref[...]
        radius = 1.0 / math.sqrt(abs(kk))
        nrm = jnp.sqrt(jnp.sum(f * f, axis=-1, keepdims=True)) + EPS
        p = f / nrm * (0.45 * radius)
        p_ref[...] = p
        ps.append(p)
    ps.append(r2[...])
    ps.append(jnp.zeros((ps[0].shape[0], 32), jnp.float32))
    pt_ref[...] = jnp.concatenate(ps, axis=-1)


_tc_prep = pl.pallas_call(
    _prep_body,
    grid=(N // _R1,),
    in_specs=[
        pl.BlockSpec((_R1, 32), lambda i: (i, 0)),
        pl.BlockSpec((_R1, 32), lambda i: (i, 0)),
        pl.BlockSpec((_R1, 32), lambda i: (i, 0)),
    ],
    out_specs=[
        pl.BlockSpec((_R1, 32), lambda i: (i, 0)),
        pl.BlockSpec((_R1, 32), lambda i: (i, 0)),
        pl.BlockSpec((_R1, 128), lambda i: (i, 0)),
    ],
    out_shape=[
        jax.ShapeDtypeStruct((N, 32), jnp.float32),
        jax.ShapeDtypeStruct((N, 32), jnp.float32),
        jax.ShapeDtypeStruct((N, 128), jnp.float32),
    ],
)


def _dis_of(dega, degb):
    deg = dega[:, 0:1] + degb[:, 0:1]
    return 1.0 / jnp.sqrt(jnp.maximum(deg, 1.0))


def _xs_body(dega, degb, x_ref, xs_ref):
    xs_ref[...] = x_ref[...] * _dis_of(dega[...], degb[...])


_tc_xs = pl.pallas_call(
    _xs_body,
    grid=(N // _R1,),
    in_specs=[
        pl.BlockSpec((_R1, 128), lambda i: (i, 0)),
        pl.BlockSpec((_R1, 128), lambda i: (i, 0)),
        pl.BlockSpec((_R1, 128), lambda i: (i, 0)),
    ],
    out_specs=pl.BlockSpec((_R1, 128), lambda i: (i, 0)),
    out_shape=jax.ShapeDtypeStruct((N, 128), jnp.float32),
)


def _layer1_body(g1a, g1b, dega, degb, w1, b1, out_ref):
    dis = _dis_of(dega[...], degb[...])
    g = (g1a[...] + g1b[...]) * dis
    h = jnp.dot(g, w1[...], preferred_element_type=jnp.float32) + b1[...]
    out_ref[...] = jnp.maximum(h, 0.0) * dis


_tc_layer1 = pl.pallas_call(
    _layer1_body,
    grid=(N // _R1,),
    in_specs=[
        pl.BlockSpec((_R1, 128), lambda i: (i, 0)),
        pl.BlockSpec((_R1, 128), lambda i: (i, 0)),
        pl.BlockSpec((_R1, 128), lambda i: (i, 0)),
        pl.BlockSpec((_R1, 128), lambda i: (i, 0)),
        pl.BlockSpec((128, 128), lambda i: (0, 0)),
        pl.BlockSpec((1, 128), lambda i: (0, 0)),
    ],
    out_specs=pl.BlockSpec((_R1, 128), lambda i: (i, 0)),
    out_shape=jax.ShapeDtypeStruct((N, 128), jnp.float32),
)


_R2 = 512
_NB = CL_B // _R2


def _lap_feats(p, w, b, kk):
    pw = lax.dot_general(p, w, (((1,), (1,)), ((), ())),
                         preferred_element_type=jnp.float32)
    if kk == 0.0:
        dist = pw
    else:
        xx = jnp.sum(p * p, axis=-1, keepdims=True)
        ww = jnp.sum(w * w, axis=-1)[None, :]
        div = xx - 2.0 * pw + ww
        dist = jnp.log((1.0 + kk * xx) / (div + EPS))
    return jnp.exp((D_FACT - 1) * dist / 2.0) * jnp.cos(dist + b)


def _cl_body(g2a, g2b, dega, degb, w2, bias2v, p0, p1, p2,
             ws0, ws1, ws2, bs0, bs1, bs2,
             rs_out, cs_out, ps_out, h4s, laps, rs, cs, ps):
    i = pl.program_id(0)
    j = pl.program_id(1)

    @pl.when(j == 0)
    def _():
        dis = _dis_of(dega[...], degb[...])
        g = (g2a[...] + g2b[...]) * dis
        h4s[...] = (jnp.dot(g, w2[...], preferred_element_type=jnp.float32)
                    + bias2v[...])

    @pl.when(i == 0)
    def _():
        laps[pl.ds(j * _R2, _R2), :] = jnp.concatenate(
            [_lap_feats(p0[...], ws0[...], bs0[...], 0.5),
             _lap_feats(p1[...], ws1[...], bs1[...], -0.3),
             _lap_feats(p2[...], ws2[...], bs2[...], 0.0)], axis=-1)

    hb = h4s[...]
    lb = laps[pl.ds(j * _R2, _R2), :]
    n1 = jnp.sqrt(jnp.sum(hb * hb, axis=-1, keepdims=True))
    n2 = jnp.sqrt(jnp.sum(lb * lb, axis=-1))[None, :]
    d = lax.dot_general(hb, lb, (((1,), (1,)), ((), ())),
                        preferred_element_type=jnp.float32)
    s = jnp.exp(d / (n1 * n2 + EPS) / TEMP)
    rowv = jnp.sum(s, axis=1)[None, :]
    colv = jnp.sum(s, axis=0)[None, :]

    @pl.when(j == 0)
    def _():
        rs[pl.ds(i, 1), :] = rowv

    @pl.when(j != 0)
    def _():
        rs[pl.ds(i, 1), :] += rowv

    @pl.when(i == 0)
    def _():
        cs[pl.ds(j, 1), :] = colv

    @pl.when(i != 0)
    def _():
        cs[pl.ds(j, 1), :] += colv

    @pl.when(i == j)
    def _():
        rr = lax.broadcasted_iota(jnp.int32, (_R2, _R2), 0)
        cc = lax.broadcasted_iota(jnp.int32, (_R2, _R2), 1)
        diag = jnp.sum(jnp.where(rr == cc, s, 0.0), axis=1)[None, :]
        ps[pl.ds(i, 1), :] = diag

    @pl.when((i == _NB - 1) & (j == _NB - 1))
    def _():
        rs_out[...] = rs[...]
        cs_out[...] = cs[...]
        ps_out[...] = ps[...]


_tc_cl = pl.pallas_call(
    _cl_body,
    grid=(_NB, _NB),
    in_specs=[
        pl.BlockSpec((_R2, 128), lambda i, j: (i, 0)),
        pl.BlockSpec((_R2, 128), lambda i, j: (i, 0)),
        pl.BlockSpec((_R2, 128), lambda i, j: (i, 0)),
        pl.BlockSpec((_R2, 128), lambda i, j: (i, 0)),
        pl.BlockSpec((128, 192), lambda i, j: (0, 0)),
        pl.BlockSpec((1, 192), lambda i, j: (0, 0)),
        pl.BlockSpec((_R2, 32), lambda i, j: (j, 0)),
        pl.BlockSpec((_R2, 32), lambda i, j: (j, 0)),
        pl.BlockSpec((_R2, 32), lambda i, j: (j, 0)),
        pl.BlockSpec((64, 32), lambda i, j: (0, 0)),
        pl.BlockSpec((64, 32), lambda i, j: (0, 0)),
        pl.BlockSpec((64, 32), lambda i, j: (0, 0)),
        pl.BlockSpec((1, 64), lambda i, j: (0, 0)),
        pl.BlockSpec((1, 64), lambda i, j: (0, 0)),
        pl.BlockSpec((1, 64), lambda i, j: (0, 0)),
    ],
    out_specs=[
        pl.BlockSpec((_NB, _R2), lambda i, j: (0, 0)),
        pl.BlockSpec((_NB, _R2), lambda i, j: (0, 0)),
        pl.BlockSpec((_NB, _R2), lambda i, j: (0, 0)),
    ],
    out_shape=[
        jax.ShapeDtypeStruct((_NB, _R2), jnp.float32),
        jax.ShapeDtypeStruct((_NB, _R2), jnp.float32),
        jax.ShapeDtypeStruct((_NB, _R2), jnp.float32),
    ],
    scratch_shapes=[
        pltpu.VMEM((_R2, 192), jnp.float32),
        pltpu.VMEM((CL_B, 192), jnp.float32),
        pltpu.VMEM((_NB, _R2), jnp.float32),
        pltpu.VMEM((_NB, _R2), jnp.float32),
        pltpu.VMEM((_NB, _R2), jnp.float32),
    ],
)


_B6 = 2000
_NST = M // _B6
_SETS = [(0, 1, 2, True), (3, 4, 2, False)]


def _motif_body(g_ref, w1_ref, b1_ref, w2r_ref, b2_ref,
                rs_ref, cs_ref, ps_ref, out_ref, acc):
    i = pl.program_id(0)

    @pl.when(i == 0)
    def _():
        acc[...] = jnp.zeros_like(acc)

    w1 = w1_ref[...]
    wa, wb, wc = w1[0:32], w1[32:64], w1[64:96]
    b1 = b1_ref[...]
    w2r = w2r_ref[...]
    b2 = b2_ref[...]
    for si, (qa, qb, qc, pos) in enumerate(_SETS):
        ga, gb, gc = g_ref[qa], g_ref[qb], g_ref[qc]
        for t in range(3):
            cols = slice(t * 32, t * 32 + 32)
            pre = (jnp.dot(ga[:, cols], wa, preferred_element_type=jnp.float32)
                   + jnp.dot(gb[:, cols], wb, preferred_element_type=jnp.float32)
                   + jnp.dot(gc[:, cols], wc, preferred_element_type=jnp.float32)
                   + b1)
            h = jnp.maximum(pre, 0.0)
            z = jnp.sum(h * w2r, axis=-1, keepdims=True) + b2
            sg = 1.0 / (1.0 + jnp.exp(-z))
            pp = jnp.clip(sg, 1e-6, 1.0 - 1e-6)
            val = -jnp.log(pp) if pos else -jnp.log(1.0 - pp)
            sidx = t * 2 + si
            acc[sidx, :] = acc[sidx, :] + jnp.sum(val)

    @pl.when(i == _NST - 1)
    def _():
        rsv = rs_ref[...]
        csv = cs_ref[...]
        psv = ps_ref[...]
        l1 = jnp.sum(-jnp.log(psv / (csv - psv) + EPS)) / float(CL_B)
        l2 = jnp.sum(-jnp.log(psv / (rsv - psv) + EPS)) / float(CL_B)
        m = acc[...][:, 0:1]
        mot = jnp.sum(m[0:6]) / float(M)
        out_ref[...] = jnp.full((1, 128), (l1 + l2) * 0.5 + mot, jnp.float32)


_tc_motif = pl.pallas_call(
    _motif_body,
    grid=(_NST,),
    in_specs=[
        pl.BlockSpec((5, _B6, 128), lambda i: (0, i, 0)),
        pl.BlockSpec((96, 64), lambda i: (0, 0)),
        pl.BlockSpec((1, 64), lambda i: (0, 0)),
        pl.BlockSpec((1, 64), lambda i: (0, 0)),
        pl.BlockSpec((1, 1), lambda i: (0, 0)),
        pl.BlockSpec((_NB, _R2), lambda i: (0, 0)),
        pl.BlockSpec((_NB, _R2), lambda i: (0, 0)),
        pl.BlockSpec((_NB, _R2), lambda i: (0, 0)),
    ],
    out_specs=pl.BlockSpec((1, 128), lambda i: (0, 0)),
    out_shape=jax.ShapeDtypeStruct((1, 128), jnp.float32),
    scratch_shapes=[pltpu.VMEM((8, 128), jnp.float32)],
)


def kernel(x, edge_index, motif, neg_motif, rm_feat0, rm_feat1, rm_feat_free,
           W1, b1, W2, b2, Ws0, Ws1, Ws2, bias0, bias1, bias2,
           mc_W1, mc_b1, mc_W2, mc_b2):
    src_flat = edge_index[0].astype(jnp.int32)
    dst_flat = edge_index[1].astype(jnp.int32)
    idxs = [motif[0].astype(jnp.int32), motif[1].astype(jnp.int32),
            motif[2].astype(jnp.int32), neg_motif[0].astype(jnp.int32),
            neg_motif[1].astype(jnp.int32)]

    ones128 = jnp.ones((ECH, 128), jnp.float32)
    zeros128 = jnp.zeros((128, 128), jnp.float32)

    p0, p1, ptab = _tc_prep(rm_feat0, rm_feat1, rm_feat_free)
    degp = _sc_deg(dst_flat, ones128, zeros128)
    dega, degb = degp[0], degp[1]
    xs = _tc_xs(dega, degb, x)
    g1 = _segsum_full(src_flat, dst_flat, xs, zeros128)
    hs = _tc_layer1(g1[0], g1[1], dega, degb, W1, b1.reshape(1, 128))
    g2 = _segsum_cl(src_flat, dst_flat, hs, zeros128)
    G = _sc_gather(ptab, idxs)
    rs, cs, ps = _tc_cl(g2[0], g2[1], dega, degb, W2, b2.reshape(1, 192),
                        p0, p1, rm_feat_free, Ws0, Ws1, Ws2,
                        bias0.reshape(1, 64), bias1.reshape(1, 64),
                        bias2.reshape(1, 64))
    loss = _tc_motif(G, mc_W1, mc_b1.reshape(1, 64),
                     mc_W2.reshape(1, 64), mc_b2.reshape(1, 1),
                     rs, cs, ps)[0, 0]
    return (p0, p1, rm_feat_free, loss)
